# Initial kernel scaffold; baseline (speedup 1.0000x reference)
import jax
import jax.numpy as jnp
from jax.experimental import pallas as pl


def kernel(x, edge_index, W0, b0, W1, b1, W2, b2, W3, b3, Wout, bout):
    raise NotImplementedError("write your pallas kernel here")


# speedup vs baseline: 9.0669x; 9.0669x over previous
import functools

import jax
import jax.numpy as jnp
from jax import lax
from jax.experimental import pallas as pl
from jax.experimental.pallas import tpu as pltpu
from jax.experimental.pallas import tpu_sc as plsc

NC = 2
NS = 16
CH = 128
F = 128


def _tile_row_copy(s, n, copy_fn):
    rpt = -(-(n // NS) // 8) * 8
    last = n - (NS - 1) * rpt
    assert last > 0 and last % 8 == 0

    @pl.when(s < NS - 1)
    def _():
        copy_fn(pl.multiple_of(s * rpt, 8), rpt)

    @pl.when(s == NS - 1)
    def _():
        copy_fn((NS - 1) * rpt, last)


@functools.lru_cache(maxsize=None)
def _make_sc_deg(n, e):
    ept = e // (NC * NS)
    n_full, rem = divmod(ept, CH)
    mesh = plsc.VectorSubcoreMesh(core_axis_name="c", subcore_axis_name="s")

    @functools.partial(
        pl.kernel,
        out_type=jax.ShapeDtypeStruct((NC, n, F), jnp.float32),
        mesh=mesh,
        scratch_types=[
            pltpu.VMEM_SHARED((n, F), jnp.float32),
            pltpu.VMEM((CH, F), jnp.float32),
            pltpu.VMEM((CH,), jnp.int32),
            pltpu.VMEM((max(rem, 8),), jnp.int32),
        ],
    )
    def sc_deg(dst_hbm, zeros_hbm, ones_hbm, deg_hbm, dacc, ones_v, didx, rdidx):
        c = lax.axis_index("c")
        s = lax.axis_index("s")
        _tile_row_copy(s, n, lambda r0, sz: pltpu.sync_copy(
            zeros_hbm.at[pl.ds(0, sz)], dacc.at[pl.ds(r0, sz)]))
        pltpu.sync_copy(ones_hbm, ones_v)
        plsc.subcore_barrier()
        base = (c * NS + s) * ept

        def body(j, carry):
            off = base + j * CH
            pltpu.sync_copy(dst_hbm.at[pl.ds(off, CH)], didx)
            pltpu.sync_copy(ones_v, dacc.at[didx], add=True)
            return carry

        lax.fori_loop(0, n_full, body, 0)
        if rem:
            off = base + n_full * CH
            pltpu.sync_copy(dst_hbm.at[pl.ds(off, rem)], rdidx.at[pl.ds(0, rem)])
            pltpu.sync_copy(ones_v.at[pl.ds(0, rem)],
                            dacc.at[rdidx.at[pl.ds(0, rem)]], add=True)
        plsc.subcore_barrier()
        _tile_row_copy(s, n, lambda r0, sz: pltpu.sync_copy(
            dacc.at[pl.ds(r0, sz)], deg_hbm.at[c, pl.ds(r0, sz)]))

    return sc_deg


@functools.lru_cache(maxsize=None)
def _make_sc_scatter(n, e):
    ept = e // NS
    n_full, rem = divmod(ept, CH)
    mesh = plsc.VectorSubcoreMesh(core_axis_name="c", subcore_axis_name="s")

    @functools.partial(
        pl.kernel,
        out_type=jax.ShapeDtypeStruct((NC, n, F), jnp.float32),
        mesh=mesh,
        scratch_types=[
            pltpu.VMEM_SHARED((n, F), jnp.float32),
            pltpu.VMEM((CH, F), jnp.float32),
            pltpu.VMEM((CH,), jnp.int32),
            pltpu.VMEM((CH,), jnp.int32),
            pltpu.VMEM((max(rem, 8), F), jnp.float32),
            pltpu.VMEM((max(rem, 8),), jnp.int32),
            pltpu.VMEM((max(rem, 8),), jnp.int32),
        ],
    )
    def sc_scatter(p_hbm, src_hbm, dst_hbm, q_hbm, acc, rows, sidx, didx,
                   rrows, rsidx, rdidx):
        c = lax.axis_index("c")
        s = lax.axis_index("s")
        _tile_row_copy(s, n, lambda r0, sz: pltpu.sync_copy(
            p_hbm.at[c, pl.ds(r0, sz)], acc.at[pl.ds(r0, sz)]))
        plsc.subcore_barrier()
        base = s * ept

        def body(j, carry):
            off = base + j * CH
            pltpu.sync_copy(src_hbm.at[pl.ds(off, CH)], sidx)
            pltpu.sync_copy(dst_hbm.at[pl.ds(off, CH)], didx)
            pltpu.sync_copy(p_hbm.at[c].at[sidx], rows)
            pltpu.sync_copy(rows, acc.at[didx], add=True)
            return carry

        lax.fori_loop(0, n_full, body, 0)
        if rem:
            off = base + n_full * CH
            pltpu.sync_copy(src_hbm.at[pl.ds(off, rem)], rsidx.at[pl.ds(0, rem)])
            pltpu.sync_copy(dst_hbm.at[pl.ds(off, rem)], rdidx.at[pl.ds(0, rem)])
            pltpu.sync_copy(p_hbm.at[c].at[rsidx.at[pl.ds(0, rem)]],
                            rrows.at[pl.ds(0, rem)])
            pltpu.sync_copy(rrows.at[pl.ds(0, rem)],
                            acc.at[rdidx.at[pl.ds(0, rem)]], add=True)
        plsc.subcore_barrier()
        _tile_row_copy(s, n, lambda r0, sz: pltpu.sync_copy(
            acc.at[pl.ds(r0, sz)], q_hbm.at[c, pl.ds(r0, sz)]))

    return sc_scatter


BN = 1000


def _tc_first_body(degp_ref, x_ref, w_ref, dis_ref, p_ref):
    deg = degp_ref[0, :, :1] + degp_ref[1, :, :1] + 1.0
    dis = lax.rsqrt(deg)
    p = jnp.dot(x_ref[...], w_ref[...], preferred_element_type=jnp.float32)
    p = p * dis
    dis_ref[...] = dis
    p_ref[0] = p[:, :F]
    p_ref[1] = p[:, F:]


def _tc_mid_body(q_ref, dis_ref, w_ref, b_ref, p_ref):
    dis = dis_ref[...]
    b = b_ref[...]
    h0 = jnp.tanh(q_ref[0] * dis + b[:, :F])
    h1 = jnp.tanh(q_ref[1] * dis + b[:, F:])
    p = (jnp.dot(h0, w_ref[0], preferred_element_type=jnp.float32)
         + jnp.dot(h1, w_ref[1], preferred_element_type=jnp.float32))
    p = p * dis
    p_ref[0] = p[:, :F]
    p_ref[1] = p[:, F:]


def _tc_last_body(q_ref, dis_ref, w_ref, b_ref, bout_ref, o_ref):
    dis = dis_ref[...]
    b = b_ref[...]
    h0 = jnp.tanh(q_ref[0] * dis + b[:, :F])
    h1 = jnp.tanh(q_ref[1] * dis + b[:, F:])
    o_ref[...] = (jnp.dot(h0, w_ref[0], preferred_element_type=jnp.float32)
                  + jnp.dot(h1, w_ref[1], preferred_element_type=jnp.float32)
                  + bout_ref[...])


def _tc_first(degp, x, w0):
    n, d_in = x.shape
    d_h = w0.shape[1]
    grid = n // BN
    return pl.pallas_call(
        _tc_first_body,
        grid=(grid,),
        in_specs=[
            pl.BlockSpec((NC, BN, F), lambda i: (0, i, 0)),
            pl.BlockSpec((BN, d_in), lambda i: (i, 0)),
            pl.BlockSpec((d_in, d_h), lambda i: (0, 0)),
        ],
        out_specs=[
            pl.BlockSpec((BN, 1), lambda i: (i, 0)),
            pl.BlockSpec((NC, BN, F), lambda i: (0, i, 0)),
        ],
        out_shape=[
            jax.ShapeDtypeStruct((n, 1), jnp.float32),
            jax.ShapeDtypeStruct((NC, n, F), jnp.float32),
        ],
    )(degp, x, w0)


def _tc_mid(q, dis, w, b):
    n = dis.shape[0]
    d_h = w.shape[2]
    grid = n // BN
    return pl.pallas_call(
        _tc_mid_body,
        grid=(grid,),
        in_specs=[
            pl.BlockSpec((NC, BN, F), lambda i: (0, i, 0)),
            pl.BlockSpec((BN, 1), lambda i: (i, 0)),
            pl.BlockSpec((NC, F, d_h), lambda i: (0, 0, 0)),
            pl.BlockSpec((1, 2 * F), lambda i: (0, 0)),
        ],
        out_specs=pl.BlockSpec((NC, BN, F), lambda i: (0, i, 0)),
        out_shape=jax.ShapeDtypeStruct((NC, n, F), jnp.float32),
    )(q, dis, w, b)


def _tc_last(q, dis, w, b, bout):
    n = dis.shape[0]
    d_out = w.shape[2]
    grid = n // BN
    return pl.pallas_call(
        _tc_last_body,
        grid=(grid,),
        in_specs=[
            pl.BlockSpec((NC, BN, F), lambda i: (0, i, 0)),
            pl.BlockSpec((BN, 1), lambda i: (i, 0)),
            pl.BlockSpec((NC, F, d_out), lambda i: (0, 0, 0)),
            pl.BlockSpec((1, 2 * F), lambda i: (0, 0)),
            pl.BlockSpec((1, d_out), lambda i: (0, 0)),
        ],
        out_specs=pl.BlockSpec((BN, d_out), lambda i: (i, 0)),
        out_shape=jax.ShapeDtypeStruct((n, d_out), jnp.float32),
    )(q, dis, w, b, bout)


def kernel(x, edge_index, W0, b0, W1, b1, W2, b2, W3, b3, Wout, bout):
    n = x.shape[0]
    e = edge_index.shape[1]

    sc_deg = _make_sc_deg(n, e)
    sc_scatter = _make_sc_scatter(n, e)

    zeros = jnp.zeros((-(-(n // NS) // 8) * 8, F), jnp.float32)
    ones = jnp.ones((CH, F), jnp.float32)
    src = edge_index[0]
    dst = edge_index[1]

    degp = sc_deg(dst, zeros, ones)
    dis, p = _tc_first(degp, x, W0)

    q = sc_scatter(p, src, dst)
    p = _tc_mid(q, dis, W1.reshape(NC, F, -1), b0.reshape(1, -1))
    q = sc_scatter(p, src, dst)
    p = _tc_mid(q, dis, W2.reshape(NC, F, -1), b1.reshape(1, -1))
    q = sc_scatter(p, src, dst)
    p = _tc_mid(q, dis, W3.reshape(NC, F, -1), b2.reshape(1, -1))
    q = sc_scatter(p, src, dst)
    return _tc_last(q, dis, Wout.reshape(NC, F, -1), b3.reshape(1, -1),
                    bout.reshape(1, -1))


# speedup vs baseline: 14.7000x; 1.6213x over previous
import functools

import jax
import jax.numpy as jnp
from jax import lax
from jax.experimental import pallas as pl
from jax.experimental.pallas import tpu as pltpu
from jax.experimental.pallas import tpu_sc as plsc

NC = 2
NS = 16
CH = 128
F = 128


def _tile_row_copy(s, n, copy_fn):
    rpt = -(-(n // NS) // 8) * 8
    last = n - (NS - 1) * rpt
    assert last > 0 and last % 8 == 0

    @pl.when(s < NS - 1)
    def _():
        copy_fn(pl.multiple_of(s * rpt, 8), rpt)

    @pl.when(s == NS - 1)
    def _():
        copy_fn((NS - 1) * rpt, last)


@functools.lru_cache(maxsize=None)
def _make_sc_deg(n, e):
    ept = e // (NC * NS)
    n_full, rem = divmod(ept, CH)
    mesh = plsc.VectorSubcoreMesh(core_axis_name="c", subcore_axis_name="s")

    @functools.partial(
        pl.kernel,
        out_type=jax.ShapeDtypeStruct((NC, n, F), jnp.float32),
        mesh=mesh,
        scratch_types=[
            pltpu.VMEM_SHARED((n, F), jnp.float32),
            pltpu.VMEM((CH, F), jnp.float32),
            pltpu.VMEM((CH,), jnp.int32),
            pltpu.VMEM((max(rem, 8),), jnp.int32),
        ],
    )
    def sc_deg(dst_hbm, zeros_hbm, ones_hbm, deg_hbm, dacc, ones_v, didx, rdidx):
        c = lax.axis_index("c")
        s = lax.axis_index("s")
        _tile_row_copy(s, n, lambda r0, sz: pltpu.sync_copy(
            zeros_hbm.at[pl.ds(0, sz)], dacc.at[pl.ds(r0, sz)]))
        pltpu.sync_copy(ones_hbm, ones_v)
        plsc.subcore_barrier()
        base = (c * NS + s) * ept

        def body(j, carry):
            off = base + j * CH
            pltpu.sync_copy(dst_hbm.at[pl.ds(off, CH)], didx)
            pltpu.sync_copy(ones_v, dacc.at[didx], add=True)
            return carry

        lax.fori_loop(0, n_full, body, 0)
        if rem:
            off = base + n_full * CH
            pltpu.sync_copy(dst_hbm.at[pl.ds(off, rem)], rdidx.at[pl.ds(0, rem)])
            pltpu.sync_copy(ones_v.at[pl.ds(0, rem)],
                            dacc.at[rdidx.at[pl.ds(0, rem)]], add=True)
        plsc.subcore_barrier()
        _tile_row_copy(s, n, lambda r0, sz: pltpu.sync_copy(
            dacc.at[pl.ds(r0, sz)], deg_hbm.at[c, pl.ds(r0, sz)]))

    return sc_deg


NB = 2
NSLOT = 2 * NB


def _edge_chunk_counts(e):
    total = e // CH
    lo, extra = divmod(total, NS)
    return lo, extra


@functools.lru_cache(maxsize=None)
def _make_sc_scatter(n, e):
    lo, extra = _edge_chunk_counts(e)
    kpt = lo + (1 if extra else 0)
    main = lo
    assert main % NSLOT == 0
    n_bodies = (main - NSLOT) // NSLOT
    mesh = plsc.VectorSubcoreMesh(core_axis_name="c", subcore_axis_name="s")

    @functools.partial(
        pl.kernel,
        out_type=jax.ShapeDtypeStruct((NC, n, F), jnp.float32),
        mesh=mesh,
        scratch_types=[
            pltpu.VMEM_SHARED((n, F), jnp.float32),
            pltpu.VMEM((NB, CH, F), jnp.float32),
            pltpu.VMEM((2 * NSLOT, CH), jnp.int32),
            [pltpu.SemaphoreType.DMA] * NB,
            [pltpu.SemaphoreType.DMA] * NB,
            [pltpu.SemaphoreType.DMA] * NSLOT,
        ],
    )
    def sc_scatter(p_hbm, ei_hbm, q_hbm, acc, rows, idxb, gsem, ssem, isem):
        c = lax.axis_index("c")
        s = lax.axis_index("s")

        def prefetch(j, slot):
            jj = jnp.minimum(j, kpt - 1)
            pltpu.async_copy(ei_hbm.at[s, jj], idxb.at[pl.ds(2 * slot, 2)],
                             isem[slot])

        def wait_idx(slot):
            pltpu.make_async_copy(ei_hbm.at[s, 0],
                                  idxb.at[pl.ds(2 * slot, 2)],
                                  isem[slot]).wait()

        def gather(slot, b):
            return pltpu.async_copy(p_hbm.at[c].at[idxb.at[2 * slot]],
                                    rows.at[b], gsem[b])

        def scatter(slot, b):
            pltpu.async_copy(rows.at[b], acc.at[idxb.at[2 * slot + 1]],
                             ssem[b], add=True)

        def drain_scatter(b):
            pltpu.make_async_copy(p_hbm.at[c, pl.ds(0, CH)], rows.at[b],
                                  ssem[b]).wait()

        for slot in range(NSLOT):
            prefetch(jnp.int32(slot), slot)
        _tile_row_copy(s, n, lambda r0, sz: pltpu.sync_copy(
            p_hbm.at[c, pl.ds(r0, sz)], acc.at[pl.ds(r0, sz)]))
        plsc.subcore_barrier()

        ds_ = []
        for i in range(NB):
            wait_idx(i)
            ds_.append(gather(i, i))
        for i in range(NB):
            ds_[i].wait()
            scatter(i, i)
        ds_ = []
        for i in range(NB):
            drain_scatter(i)
            prefetch(jnp.int32(NSLOT + i), i)
            wait_idx(NB + i)
            ds_.append(gather(NB + i, i))
        for i in range(NB):
            ds_[i].wait()
            scatter(NB + i, i)

        def body(m, carry):
            jb = NSLOT + m * NSLOT
            for g in range(2):
                ds_ = []
                for i in range(NB):
                    t = g * NB + i
                    drain_scatter(i)
                    prefetch(jb + t + NB, (t + NB) % NSLOT)
                    wait_idx(t)
                    ds_.append(gather(t, i))
                for i in range(NB):
                    ds_[i].wait()
                    scatter(g * NB + i, i)
            return carry

        lax.fori_loop(0, n_bodies, body, 0)

        for i in range(NB):
            drain_scatter(i)
        wait_idx(0)
        if extra:
            @pl.when(s < extra)
            def _():
                pltpu.sync_copy(p_hbm.at[c].at[idxb.at[0]], rows.at[0])
                pltpu.sync_copy(rows.at[0], acc.at[idxb.at[1]], add=True)
        for i in range(1, NB):
            wait_idx(i)

        plsc.subcore_barrier()
        _tile_row_copy(s, n, lambda r0, sz: pltpu.sync_copy(
            acc.at[pl.ds(r0, sz)], q_hbm.at[c, pl.ds(r0, sz)]))

    return sc_scatter


BN = 1000


def _tc_first_body(degp_ref, x_ref, w_ref, dis_ref, p_ref):
    deg = degp_ref[0, :, :1] + degp_ref[1, :, :1] + 1.0
    dis = lax.rsqrt(deg)
    p = jnp.dot(x_ref[...], w_ref[...], preferred_element_type=jnp.float32)
    p = p * dis
    dis_ref[...] = dis
    p_ref[0] = p[:, :F]
    p_ref[1] = p[:, F:]


def _tc_mid_body(q_ref, dis_ref, w_ref, b_ref, p_ref):
    dis = dis_ref[...]
    b = b_ref[...]
    h0 = jnp.tanh(q_ref[0] * dis + b[:, :F])
    h1 = jnp.tanh(q_ref[1] * dis + b[:, F:])
    p = (jnp.dot(h0, w_ref[0], preferred_element_type=jnp.float32)
         + jnp.dot(h1, w_ref[1], preferred_element_type=jnp.float32))
    p = p * dis
    p_ref[0] = p[:, :F]
    p_ref[1] = p[:, F:]


def _tc_last_body(q_ref, dis_ref, w_ref, b_ref, bout_ref, o_ref):
    dis = dis_ref[...]
    b = b_ref[...]
    h0 = jnp.tanh(q_ref[0] * dis + b[:, :F])
    h1 = jnp.tanh(q_ref[1] * dis + b[:, F:])
    o_ref[...] = (jnp.dot(h0, w_ref[0], preferred_element_type=jnp.float32)
                  + jnp.dot(h1, w_ref[1], preferred_element_type=jnp.float32)
                  + bout_ref[...])


def _tc_first(degp, x, w0):
    n, d_in = x.shape
    d_h = w0.shape[1]
    grid = n // BN
    return pl.pallas_call(
        _tc_first_body,
        grid=(grid,),
        in_specs=[
            pl.BlockSpec((NC, BN, F), lambda i: (0, i, 0)),
            pl.BlockSpec((BN, d_in), lambda i: (i, 0)),
            pl.BlockSpec((d_in, d_h), lambda i: (0, 0)),
        ],
        out_specs=[
            pl.BlockSpec((BN, 1), lambda i: (i, 0)),
            pl.BlockSpec((NC, BN, F), lambda i: (0, i, 0)),
        ],
        out_shape=[
            jax.ShapeDtypeStruct((n, 1), jnp.float32),
            jax.ShapeDtypeStruct((NC, n, F), jnp.float32),
        ],
    )(degp, x, w0)


def _tc_mid(q, dis, w, b):
    n = dis.shape[0]
    d_h = w.shape[2]
    grid = n // BN
    return pl.pallas_call(
        _tc_mid_body,
        grid=(grid,),
        in_specs=[
            pl.BlockSpec((NC, BN, F), lambda i: (0, i, 0)),
            pl.BlockSpec((BN, 1), lambda i: (i, 0)),
            pl.BlockSpec((NC, F, d_h), lambda i: (0, 0, 0)),
            pl.BlockSpec((1, 2 * F), lambda i: (0, 0)),
        ],
        out_specs=pl.BlockSpec((NC, BN, F), lambda i: (0, i, 0)),
        out_shape=jax.ShapeDtypeStruct((NC, n, F), jnp.float32),
    )(q, dis, w, b)


def _tc_last(q, dis, w, b, bout):
    n = dis.shape[0]
    d_out = w.shape[2]
    grid = n // BN
    return pl.pallas_call(
        _tc_last_body,
        grid=(grid,),
        in_specs=[
            pl.BlockSpec((NC, BN, F), lambda i: (0, i, 0)),
            pl.BlockSpec((BN, 1), lambda i: (i, 0)),
            pl.BlockSpec((NC, F, d_out), lambda i: (0, 0, 0)),
            pl.BlockSpec((1, 2 * F), lambda i: (0, 0)),
            pl.BlockSpec((1, d_out), lambda i: (0, 0)),
        ],
        out_specs=pl.BlockSpec((BN, d_out), lambda i: (i, 0)),
        out_shape=jax.ShapeDtypeStruct((n, d_out), jnp.float32),
    )(q, dis, w, b, bout)


def kernel(x, edge_index, W0, b0, W1, b1, W2, b2, W3, b3, Wout, bout):
    n = x.shape[0]
    e = edge_index.shape[1]

    sc_deg = _make_sc_deg(n, e)
    sc_scatter = _make_sc_scatter(n, e)

    zeros = jnp.zeros((-(-(n // NS) // 8) * 8, F), jnp.float32)
    ones = jnp.ones((CH, F), jnp.float32)
    src = edge_index[0]
    dst = edge_index[1]

    lo, extra = _edge_chunk_counts(e)

    def chunked(a):
        if not extra:
            return a.reshape(NS, lo, CH)
        p1 = a[:extra * (lo + 1) * CH].reshape(extra, lo + 1, CH)
        p2 = a[extra * (lo + 1) * CH:].reshape(NS - extra, lo, CH)
        pad = jnp.zeros((NS - extra, 1, CH), jnp.int32)
        return jnp.concatenate([p1, jnp.concatenate([p2, pad], axis=1)], axis=0)

    ei4 = jnp.stack([chunked(src), chunked(dst)], axis=2)

    degp = sc_deg(dst, zeros, ones)
    dis, p = _tc_first(degp, x, W0)

    q = sc_scatter(p, ei4)
    p = _tc_mid(q, dis, W1.reshape(NC, F, -1), b0.reshape(1, -1))
    q = sc_scatter(p, ei4)
    p = _tc_mid(q, dis, W2.reshape(NC, F, -1), b1.reshape(1, -1))
    q = sc_scatter(p, ei4)
    p = _tc_mid(q, dis, W3.reshape(NC, F, -1), b2.reshape(1, -1))
    q = sc_scatter(p, ei4)
    return _tc_last(q, dis, Wout.reshape(NC, F, -1), b3.reshape(1, -1),
                    bout.reshape(1, -1))


# speedup vs baseline: 17.3853x; 1.1827x over previous
import functools

import jax
import jax.numpy as jnp
from jax import lax
from jax.experimental import pallas as pl
from jax.experimental.pallas import tpu as pltpu
from jax.experimental.pallas import tpu_sc as plsc

NC = 2
NS = 16
CH = 128
F = 128


def _tile_row_copy(s, n, copy_fn):
    rpt = -(-(n // NS) // 8) * 8
    last = n - (NS - 1) * rpt
    assert last > 0 and last % 8 == 0

    @pl.when(s < NS - 1)
    def _():
        copy_fn(pl.multiple_of(s * rpt, 8), rpt)

    @pl.when(s == NS - 1)
    def _():
        copy_fn((NS - 1) * rpt, last)


@functools.lru_cache(maxsize=None)
def _make_sc_deg(n, e):
    ept = e // (NC * NS)
    n_full, rem = divmod(ept, CH)
    mesh = plsc.VectorSubcoreMesh(core_axis_name="c", subcore_axis_name="s")

    @functools.partial(
        pl.kernel,
        out_type=jax.ShapeDtypeStruct((NC, n, F), jnp.float32),
        mesh=mesh,
        scratch_types=[
            pltpu.VMEM_SHARED((n, F), jnp.float32),
            pltpu.VMEM((CH, F), jnp.float32),
            pltpu.VMEM((CH,), jnp.int32),
            pltpu.VMEM((max(rem, 8),), jnp.int32),
        ],
    )
    def sc_deg(dst_hbm, zeros_hbm, ones_hbm, deg_hbm, dacc, ones_v, didx, rdidx):
        c = lax.axis_index("c")
        s = lax.axis_index("s")
        _tile_row_copy(s, n, lambda r0, sz: pltpu.sync_copy(
            zeros_hbm.at[pl.ds(0, sz)], dacc.at[pl.ds(r0, sz)]))
        pltpu.sync_copy(ones_hbm, ones_v)
        plsc.subcore_barrier()
        base = (c * NS + s) * ept

        def body(j, carry):
            off = base + j * CH
            pltpu.sync_copy(dst_hbm.at[pl.ds(off, CH)], didx)
            pltpu.sync_copy(ones_v, dacc.at[didx], add=True)
            return carry

        lax.fori_loop(0, n_full, body, 0)
        if rem:
            off = base + n_full * CH
            pltpu.sync_copy(dst_hbm.at[pl.ds(off, rem)], rdidx.at[pl.ds(0, rem)])
            pltpu.sync_copy(ones_v.at[pl.ds(0, rem)],
                            dacc.at[rdidx.at[pl.ds(0, rem)]], add=True)
        plsc.subcore_barrier()
        _tile_row_copy(s, n, lambda r0, sz: pltpu.sync_copy(
            dacc.at[pl.ds(r0, sz)], deg_hbm.at[c, pl.ds(r0, sz)]))

    return sc_deg


NB = 3
NSLOT = 2 * NB


def _edge_chunk_counts(e):
    total = e // CH
    lo, extra = divmod(total, NS)
    return lo, extra


@functools.lru_cache(maxsize=None)
def _make_sc_scatter(n, e):
    lo, extra = _edge_chunk_counts(e)
    kpt = lo + (1 if extra else 0)
    main = lo
    assert main % NSLOT == 0
    n_bodies = (main - NSLOT) // NSLOT
    mesh = plsc.VectorSubcoreMesh(core_axis_name="c", subcore_axis_name="s")

    @functools.partial(
        pl.kernel,
        out_type=jax.ShapeDtypeStruct((NC, n, F), jnp.float32),
        mesh=mesh,
        scratch_types=[
            pltpu.VMEM_SHARED((n, F), jnp.float32),
            pltpu.VMEM((NB, CH, F), jnp.float32),
            pltpu.VMEM((2 * NSLOT * CH,), jnp.int32),
            [pltpu.SemaphoreType.DMA] * NB,
            [pltpu.SemaphoreType.DMA] * NB,
            [pltpu.SemaphoreType.DMA] * NSLOT,
        ],
    )
    def sc_scatter(p_hbm, ei_hbm, q_hbm, acc, rows, idxb, gsem, ssem, isem):
        c = lax.axis_index("c")
        s = lax.axis_index("s")

        def islice(slot):
            return idxb.at[pl.ds(2 * slot * CH, 2 * CH)]

        def prefetch(j, slot):
            jj = jnp.minimum(j, kpt - 1)
            pltpu.async_copy(ei_hbm.at[s, jj], islice(slot), isem[slot])

        def wait_idx(slot):
            pltpu.make_async_copy(ei_hbm.at[s, 0], islice(slot),
                                  isem[slot]).wait()

        def gather(slot, b):
            return pltpu.async_copy(
                p_hbm.at[c].at[idxb.at[pl.ds(2 * slot * CH, CH)]],
                rows.at[b], gsem[b])

        def scatter(slot, b):
            pltpu.async_copy(rows.at[b],
                             acc.at[idxb.at[pl.ds((2 * slot + 1) * CH, CH)]],
                             ssem[b], add=True)

        def drain_scatter(b):
            pltpu.make_async_copy(p_hbm.at[c, pl.ds(0, CH)], rows.at[b],
                                  ssem[b]).wait()

        for slot in range(NSLOT):
            prefetch(jnp.int32(slot), slot)
        _tile_row_copy(s, n, lambda r0, sz: pltpu.sync_copy(
            p_hbm.at[c, pl.ds(r0, sz)], acc.at[pl.ds(r0, sz)]))
        plsc.subcore_barrier()

        ds_ = []
        for i in range(NB):
            wait_idx(i)
            ds_.append(gather(i, i))
        for i in range(NB):
            ds_[i].wait()
            scatter(i, i)
        ds_ = []
        for i in range(NB):
            drain_scatter(i)
            prefetch(jnp.int32(NSLOT + i), i)
            wait_idx(NB + i)
            ds_.append(gather(NB + i, i))
        for i in range(NB):
            ds_[i].wait()
            scatter(NB + i, i)

        def body(m, carry):
            jb = NSLOT + m * NSLOT
            for g in range(2):
                ds_ = []
                for i in range(NB):
                    t = g * NB + i
                    drain_scatter(i)
                    prefetch(jb + t + NB, (t + NB) % NSLOT)
                    wait_idx(t)
                    ds_.append(gather(t, i))
                for i in range(NB):
                    ds_[i].wait()
                    scatter(g * NB + i, i)
            return carry

        lax.fori_loop(0, n_bodies, body, 0)

        for i in range(NB):
            drain_scatter(i)
        wait_idx(0)
        if extra:
            @pl.when(s < extra)
            def _():
                pltpu.sync_copy(p_hbm.at[c].at[idxb.at[pl.ds(0, CH)]],
                                rows.at[0])
                pltpu.sync_copy(rows.at[0], acc.at[idxb.at[pl.ds(CH, CH)]],
                                add=True)
        for i in range(1, NB):
            wait_idx(i)

        plsc.subcore_barrier()
        _tile_row_copy(s, n, lambda r0, sz: pltpu.sync_copy(
            acc.at[pl.ds(r0, sz)], q_hbm.at[c, pl.ds(r0, sz)]))

    return sc_scatter


BN = 1000


def _tc_first_body(degp_ref, x_ref, w_ref, dis_ref, p_ref):
    deg = degp_ref[0, :, :1] + degp_ref[1, :, :1] + 1.0
    dis = lax.rsqrt(deg)
    p = jnp.dot(x_ref[...], w_ref[...], preferred_element_type=jnp.float32)
    p = p * dis
    dis_ref[...] = dis
    p_ref[0] = p[:, :F]
    p_ref[1] = p[:, F:]


def _tc_mid_body(q_ref, dis_ref, w_ref, b_ref, p_ref):
    dis = dis_ref[...]
    b = b_ref[...]
    h0 = jnp.tanh(q_ref[0] * dis + b[:, :F])
    h1 = jnp.tanh(q_ref[1] * dis + b[:, F:])
    p = (jnp.dot(h0, w_ref[0], preferred_element_type=jnp.float32)
         + jnp.dot(h1, w_ref[1], preferred_element_type=jnp.float32))
    p = p * dis
    p_ref[0] = p[:, :F]
    p_ref[1] = p[:, F:]


def _tc_last_body(q_ref, dis_ref, w_ref, b_ref, bout_ref, o_ref):
    dis = dis_ref[...]
    b = b_ref[...]
    h0 = jnp.tanh(q_ref[0] * dis + b[:, :F])
    h1 = jnp.tanh(q_ref[1] * dis + b[:, F:])
    o_ref[...] = (jnp.dot(h0, w_ref[0], preferred_element_type=jnp.float32)
                  + jnp.dot(h1, w_ref[1], preferred_element_type=jnp.float32)
                  + bout_ref[...])


def _tc_first(degp, x, w0):
    n, d_in = x.shape
    d_h = w0.shape[1]
    grid = n // BN
    return pl.pallas_call(
        _tc_first_body,
        grid=(grid,),
        in_specs=[
            pl.BlockSpec((NC, BN, F), lambda i: (0, i, 0)),
            pl.BlockSpec((BN, d_in), lambda i: (i, 0)),
            pl.BlockSpec((d_in, d_h), lambda i: (0, 0)),
        ],
        out_specs=[
            pl.BlockSpec((BN, 1), lambda i: (i, 0)),
            pl.BlockSpec((NC, BN, F), lambda i: (0, i, 0)),
        ],
        out_shape=[
            jax.ShapeDtypeStruct((n, 1), jnp.float32),
            jax.ShapeDtypeStruct((NC, n, F), jnp.float32),
        ],
    )(degp, x, w0)


def _tc_mid(q, dis, w, b):
    n = dis.shape[0]
    d_h = w.shape[2]
    grid = n // BN
    return pl.pallas_call(
        _tc_mid_body,
        grid=(grid,),
        in_specs=[
            pl.BlockSpec((NC, BN, F), lambda i: (0, i, 0)),
            pl.BlockSpec((BN, 1), lambda i: (i, 0)),
            pl.BlockSpec((NC, F, d_h), lambda i: (0, 0, 0)),
            pl.BlockSpec((1, 2 * F), lambda i: (0, 0)),
        ],
        out_specs=pl.BlockSpec((NC, BN, F), lambda i: (0, i, 0)),
        out_shape=jax.ShapeDtypeStruct((NC, n, F), jnp.float32),
    )(q, dis, w, b)


def _tc_last(q, dis, w, b, bout):
    n = dis.shape[0]
    d_out = w.shape[2]
    grid = n // BN
    return pl.pallas_call(
        _tc_last_body,
        grid=(grid,),
        in_specs=[
            pl.BlockSpec((NC, BN, F), lambda i: (0, i, 0)),
            pl.BlockSpec((BN, 1), lambda i: (i, 0)),
            pl.BlockSpec((NC, F, d_out), lambda i: (0, 0, 0)),
            pl.BlockSpec((1, 2 * F), lambda i: (0, 0)),
            pl.BlockSpec((1, d_out), lambda i: (0, 0)),
        ],
        out_specs=pl.BlockSpec((BN, d_out), lambda i: (i, 0)),
        out_shape=jax.ShapeDtypeStruct((n, d_out), jnp.float32),
    )(q, dis, w, b, bout)


def kernel(x, edge_index, W0, b0, W1, b1, W2, b2, W3, b3, Wout, bout):
    n = x.shape[0]
    e = edge_index.shape[1]

    sc_deg = _make_sc_deg(n, e)
    sc_scatter = _make_sc_scatter(n, e)

    zeros = jnp.zeros((-(-(n // NS) // 8) * 8, F), jnp.float32)
    ones = jnp.ones((CH, F), jnp.float32)
    src = edge_index[0]
    dst = edge_index[1]

    lo, extra = _edge_chunk_counts(e)

    def chunked(a):
        if not extra:
            return a.reshape(NS, lo, CH)
        p1 = a[:extra * (lo + 1) * CH].reshape(extra, lo + 1, CH)
        p2 = a[extra * (lo + 1) * CH:].reshape(NS - extra, lo, CH)
        pad = jnp.zeros((NS - extra, 1, CH), jnp.int32)
        return jnp.concatenate([p1, jnp.concatenate([p2, pad], axis=1)], axis=0)

    ei4 = jnp.stack([chunked(src), chunked(dst)], axis=2).reshape(
        NS, -1, 2 * CH)

    degp = sc_deg(dst, zeros, ones)
    dis, p = _tc_first(degp, x, W0)

    q = sc_scatter(p, ei4)
    p = _tc_mid(q, dis, W1.reshape(NC, F, -1), b0.reshape(1, -1))
    q = sc_scatter(p, ei4)
    p = _tc_mid(q, dis, W2.reshape(NC, F, -1), b1.reshape(1, -1))
    q = sc_scatter(p, ei4)
    p = _tc_mid(q, dis, W3.reshape(NC, F, -1), b2.reshape(1, -1))
    q = sc_scatter(p, ei4)
    return _tc_last(q, dis, Wout.reshape(NC, F, -1), b3.reshape(1, -1),
                    bout.reshape(1, -1))


# speedup vs baseline: 18.1135x; 1.0419x over previous
import functools

import jax
import jax.numpy as jnp
from jax import lax
from jax.experimental import pallas as pl
from jax.experimental.pallas import tpu as pltpu
from jax.experimental.pallas import tpu_sc as plsc

NC = 2
NS = 16
CH = 128
F = 128


def _tile_row_copy(s, n, copy_fn):
    rpt = -(-(n // NS) // 8) * 8
    last = n - (NS - 1) * rpt
    assert last > 0 and last % 8 == 0

    @pl.when(s < NS - 1)
    def _():
        copy_fn(pl.multiple_of(s * rpt, 8), rpt)

    @pl.when(s == NS - 1)
    def _():
        copy_fn((NS - 1) * rpt, last)


DSLOT = 6


@functools.lru_cache(maxsize=None)
def _make_sc_deg(n, e):
    ept = e // (NC * NS)
    n_full, rem = divmod(ept, CH)
    assert n_full % DSLOT == 0
    mesh = plsc.VectorSubcoreMesh(core_axis_name="c", subcore_axis_name="s")

    @functools.partial(
        pl.kernel,
        out_type=jax.ShapeDtypeStruct((NC, n, F), jnp.float32),
        mesh=mesh,
        scratch_types=[
            pltpu.VMEM_SHARED((n, F), jnp.float32),
            pltpu.VMEM((CH, F), jnp.float32),
            pltpu.VMEM((DSLOT * CH,), jnp.int32),
            pltpu.VMEM((max(rem, 8),), jnp.int32),
            [pltpu.SemaphoreType.DMA] * DSLOT,
            [pltpu.SemaphoreType.DMA] * DSLOT,
        ],
    )
    def sc_deg(dst_hbm, zeros_hbm, ones_hbm, deg_hbm, dacc, ones_v, idxd,
               rdidx, isem, ssem):
        c = lax.axis_index("c")
        s = lax.axis_index("s")
        base = (c * NS + s) * ept

        def islice(slot):
            return idxd.at[pl.ds(slot * CH, CH)]

        def prefetch(j, slot):
            off = base + jnp.minimum(j, n_full - 1) * CH
            pltpu.async_copy(dst_hbm.at[pl.ds(off, CH)], islice(slot),
                             isem[slot])

        def wait_idx(slot):
            pltpu.make_async_copy(dst_hbm.at[pl.ds(base, CH)], islice(slot),
                                  isem[slot]).wait()

        def scatter(slot):
            pltpu.async_copy(ones_v, dacc.at[islice(slot)], ssem[slot],
                             add=True)

        def drain_scatter(slot):
            pltpu.make_async_copy(ones_hbm, ones_v, ssem[slot]).wait()

        for slot in range(DSLOT):
            prefetch(jnp.int32(slot), slot)
        pltpu.sync_copy(ones_hbm, ones_v)
        _tile_row_copy(s, n, lambda r0, sz: pltpu.sync_copy(
            zeros_hbm.at[pl.ds(0, sz)], dacc.at[pl.ds(r0, sz)]))
        plsc.subcore_barrier()

        half = DSLOT // 2
        for t in range(half):
            wait_idx(t)
            scatter(t)
        for t in range(half, DSLOT):
            drain_scatter((t + half) % DSLOT)
            prefetch(jnp.int32(t + half), (t + half) % DSLOT)
            wait_idx(t)
            scatter(t)

        def body(m, carry):
            jb = DSLOT + m * DSLOT
            for t in range(DSLOT):
                drain_scatter((t + half) % DSLOT)
                prefetch(jb + t + half, (t + half) % DSLOT)
                wait_idx(t)
                scatter(t)
            return carry

        lax.fori_loop(0, (n_full - DSLOT) // DSLOT, body, 0)

        for t in range(half):
            wait_idx(t)
        for t in range(half, DSLOT):
            drain_scatter(t)
        if rem:
            off = base + n_full * CH
            pltpu.sync_copy(dst_hbm.at[pl.ds(off, rem)], rdidx.at[pl.ds(0, rem)])
            pltpu.sync_copy(ones_v.at[pl.ds(0, rem)],
                            dacc.at[rdidx.at[pl.ds(0, rem)]], add=True)
        plsc.subcore_barrier()
        _tile_row_copy(s, n, lambda r0, sz: pltpu.sync_copy(
            dacc.at[pl.ds(r0, sz)], deg_hbm.at[c, pl.ds(r0, sz)]))

    return sc_deg


NB = 3
NSLOT = 2 * NB


def _edge_chunk_counts(e):
    total = e // CH
    lo, extra = divmod(total, NS)
    return lo, extra


@functools.lru_cache(maxsize=None)
def _make_sc_scatter(n, e):
    lo, extra = _edge_chunk_counts(e)
    kpt = lo + (1 if extra else 0)
    main = lo
    assert main % NSLOT == 0
    n_bodies = (main - NSLOT) // NSLOT
    mesh = plsc.VectorSubcoreMesh(core_axis_name="c", subcore_axis_name="s")

    @functools.partial(
        pl.kernel,
        out_type=jax.ShapeDtypeStruct((NC, n, F), jnp.float32),
        mesh=mesh,
        scratch_types=[
            pltpu.VMEM_SHARED((n, F), jnp.float32),
            pltpu.VMEM((NB, CH, F), jnp.float32),
            pltpu.VMEM((2 * NSLOT * CH,), jnp.int32),
            [pltpu.SemaphoreType.DMA] * NB,
            [pltpu.SemaphoreType.DMA] * NB,
            [pltpu.SemaphoreType.DMA] * NSLOT,
        ],
    )
    def sc_scatter(p_hbm, ei_hbm, q_hbm, acc, rows, idxb, gsem, ssem, isem):
        c = lax.axis_index("c")
        s = lax.axis_index("s")

        def islice(slot):
            return idxb.at[pl.ds(2 * slot * CH, 2 * CH)]

        def prefetch(j, slot):
            jj = jnp.minimum(j, kpt - 1)
            pltpu.async_copy(ei_hbm.at[s, jj], islice(slot), isem[slot])

        def wait_idx(slot):
            pltpu.make_async_copy(ei_hbm.at[s, 0], islice(slot),
                                  isem[slot]).wait()

        def gather(slot, b):
            return pltpu.async_copy(
                p_hbm.at[c].at[idxb.at[pl.ds(2 * slot * CH, CH)]],
                rows.at[b], gsem[b])

        def scatter(slot, b):
            pltpu.async_copy(rows.at[b],
                             acc.at[idxb.at[pl.ds((2 * slot + 1) * CH, CH)]],
                             ssem[b], add=True)

        def drain_scatter(b):
            pltpu.make_async_copy(p_hbm.at[c, pl.ds(0, CH)], rows.at[b],
                                  ssem[b]).wait()

        for slot in range(NSLOT):
            prefetch(jnp.int32(slot), slot)
        _tile_row_copy(s, n, lambda r0, sz: pltpu.sync_copy(
            p_hbm.at[c, pl.ds(r0, sz)], acc.at[pl.ds(r0, sz)]))
        plsc.subcore_barrier()

        ds_ = []
        for i in range(NB):
            wait_idx(i)
            ds_.append(gather(i, i))
        for i in range(NB):
            ds_[i].wait()
            scatter(i, i)
        ds_ = []
        for i in range(NB):
            drain_scatter(i)
            prefetch(jnp.int32(NSLOT + i), i)
            wait_idx(NB + i)
            ds_.append(gather(NB + i, i))
        for i in range(NB):
            ds_[i].wait()
            scatter(NB + i, i)

        def body(m, carry):
            jb = NSLOT + m * NSLOT
            for g in range(2):
                ds_ = []
                for i in range(NB):
                    t = g * NB + i
                    drain_scatter(i)
                    prefetch(jb + t + NB, (t + NB) % NSLOT)
                    wait_idx(t)
                    ds_.append(gather(t, i))
                for i in range(NB):
                    ds_[i].wait()
                    scatter(g * NB + i, i)
            return carry

        lax.fori_loop(0, n_bodies, body, 0)

        for i in range(NB):
            drain_scatter(i)
        wait_idx(0)
        if extra:
            @pl.when(s < extra)
            def _():
                pltpu.sync_copy(p_hbm.at[c].at[idxb.at[pl.ds(0, CH)]],
                                rows.at[0])
                pltpu.sync_copy(rows.at[0], acc.at[idxb.at[pl.ds(CH, CH)]],
                                add=True)
        for i in range(1, NB):
            wait_idx(i)

        plsc.subcore_barrier()
        _tile_row_copy(s, n, lambda r0, sz: pltpu.sync_copy(
            acc.at[pl.ds(r0, sz)], q_hbm.at[c, pl.ds(r0, sz)]))

    return sc_scatter


BN = 1000


def _tc_first_body(degp_ref, x_ref, w_ref, dis_ref, p_ref):
    deg = degp_ref[0, :, :1] + degp_ref[1, :, :1] + 1.0
    dis = lax.rsqrt(deg)
    p = jnp.dot(x_ref[...], w_ref[...], preferred_element_type=jnp.float32)
    p = p * dis
    dis_ref[...] = dis
    p_ref[0] = p[:, :F]
    p_ref[1] = p[:, F:]


def _tc_mid_body(q_ref, dis_ref, w_ref, b_ref, p_ref):
    dis = dis_ref[...]
    b = b_ref[...]
    h0 = jnp.tanh(q_ref[0] * dis + b[:, :F])
    h1 = jnp.tanh(q_ref[1] * dis + b[:, F:])
    p = (jnp.dot(h0, w_ref[0], preferred_element_type=jnp.float32)
         + jnp.dot(h1, w_ref[1], preferred_element_type=jnp.float32))
    p = p * dis
    p_ref[0] = p[:, :F]
    p_ref[1] = p[:, F:]


def _tc_last_body(q_ref, dis_ref, w_ref, b_ref, bout_ref, o_ref):
    dis = dis_ref[...]
    b = b_ref[...]
    h0 = jnp.tanh(q_ref[0] * dis + b[:, :F])
    h1 = jnp.tanh(q_ref[1] * dis + b[:, F:])
    o_ref[...] = (jnp.dot(h0, w_ref[0], preferred_element_type=jnp.float32)
                  + jnp.dot(h1, w_ref[1], preferred_element_type=jnp.float32)
                  + bout_ref[...])


def _tc_first(degp, x, w0):
    n, d_in = x.shape
    d_h = w0.shape[1]
    grid = n // BN
    return pl.pallas_call(
        _tc_first_body,
        grid=(grid,),
        in_specs=[
            pl.BlockSpec((NC, BN, F), lambda i: (0, i, 0)),
            pl.BlockSpec((BN, d_in), lambda i: (i, 0)),
            pl.BlockSpec((d_in, d_h), lambda i: (0, 0)),
        ],
        out_specs=[
            pl.BlockSpec((BN, 1), lambda i: (i, 0)),
            pl.BlockSpec((NC, BN, F), lambda i: (0, i, 0)),
        ],
        out_shape=[
            jax.ShapeDtypeStruct((n, 1), jnp.float32),
            jax.ShapeDtypeStruct((NC, n, F), jnp.float32),
        ],
    )(degp, x, w0)


def _tc_mid(q, dis, w, b):
    n = dis.shape[0]
    d_h = w.shape[2]
    grid = n // BN
    return pl.pallas_call(
        _tc_mid_body,
        grid=(grid,),
        in_specs=[
            pl.BlockSpec((NC, BN, F), lambda i: (0, i, 0)),
            pl.BlockSpec((BN, 1), lambda i: (i, 0)),
            pl.BlockSpec((NC, F, d_h), lambda i: (0, 0, 0)),
            pl.BlockSpec((1, 2 * F), lambda i: (0, 0)),
        ],
        out_specs=pl.BlockSpec((NC, BN, F), lambda i: (0, i, 0)),
        out_shape=jax.ShapeDtypeStruct((NC, n, F), jnp.float32),
    )(q, dis, w, b)


def _tc_last(q, dis, w, b, bout):
    n = dis.shape[0]
    d_out = w.shape[2]
    grid = n // BN
    return pl.pallas_call(
        _tc_last_body,
        grid=(grid,),
        in_specs=[
            pl.BlockSpec((NC, BN, F), lambda i: (0, i, 0)),
            pl.BlockSpec((BN, 1), lambda i: (i, 0)),
            pl.BlockSpec((NC, F, d_out), lambda i: (0, 0, 0)),
            pl.BlockSpec((1, 2 * F), lambda i: (0, 0)),
            pl.BlockSpec((1, d_out), lambda i: (0, 0)),
        ],
        out_specs=pl.BlockSpec((BN, d_out), lambda i: (i, 0)),
        out_shape=jax.ShapeDtypeStruct((n, d_out), jnp.float32),
    )(q, dis, w, b, bout)


def kernel(x, edge_index, W0, b0, W1, b1, W2, b2, W3, b3, Wout, bout):
    n = x.shape[0]
    e = edge_index.shape[1]

    sc_deg = _make_sc_deg(n, e)
    sc_scatter = _make_sc_scatter(n, e)

    zeros = jnp.zeros((-(-(n // NS) // 8) * 8, F), jnp.float32)
    ones = jnp.ones((CH, F), jnp.float32)
    src = edge_index[0]
    dst = edge_index[1]

    lo, extra = _edge_chunk_counts(e)

    def chunked(a):
        if not extra:
            return a.reshape(NS, lo, CH)
        p1 = a[:extra * (lo + 1) * CH].reshape(extra, lo + 1, CH)
        p2 = a[extra * (lo + 1) * CH:].reshape(NS - extra, lo, CH)
        pad = jnp.zeros((NS - extra, 1, CH), jnp.int32)
        return jnp.concatenate([p1, jnp.concatenate([p2, pad], axis=1)], axis=0)

    ei4 = jnp.stack([chunked(src), chunked(dst)], axis=2).reshape(
        NS, -1, 2 * CH)

    degp = sc_deg(dst, zeros, ones)
    dis, p = _tc_first(degp, x, W0)

    q = sc_scatter(p, ei4)
    p = _tc_mid(q, dis, W1.reshape(NC, F, -1), b0.reshape(1, -1))
    q = sc_scatter(p, ei4)
    p = _tc_mid(q, dis, W2.reshape(NC, F, -1), b1.reshape(1, -1))
    q = sc_scatter(p, ei4)
    p = _tc_mid(q, dis, W3.reshape(NC, F, -1), b2.reshape(1, -1))
    q = sc_scatter(p, ei4)
    return _tc_last(q, dis, Wout.reshape(NC, F, -1), b3.reshape(1, -1),
                    bout.reshape(1, -1))


# speedup vs baseline: 18.1502x; 1.0020x over previous
import functools

import jax
import jax.numpy as jnp
from jax import lax
from jax.experimental import pallas as pl
from jax.experimental.pallas import tpu as pltpu
from jax.experimental.pallas import tpu_sc as plsc

NC = 2
NS = 16
CH = 128
F = 128


def _tile_row_copy(s, n, copy_fn):
    rpt = -(-(n // NS) // 8) * 8
    last = n - (NS - 1) * rpt
    assert last > 0 and last % 8 == 0

    @pl.when(s < NS - 1)
    def _():
        copy_fn(pl.multiple_of(s * rpt, 8), rpt)

    @pl.when(s == NS - 1)
    def _():
        copy_fn((NS - 1) * rpt, last)


DSLOT = 6


@functools.lru_cache(maxsize=None)
def _make_sc_deg(n, e):
    ept = e // (NC * NS)
    n_full, rem = divmod(ept, CH)
    assert n_full % DSLOT == 0
    mesh = plsc.VectorSubcoreMesh(core_axis_name="c", subcore_axis_name="s")

    @functools.partial(
        pl.kernel,
        out_type=jax.ShapeDtypeStruct((NC, n, F), jnp.float32),
        mesh=mesh,
        scratch_types=[
            pltpu.VMEM_SHARED((n, F), jnp.float32),
            pltpu.VMEM((CH, F), jnp.float32),
            pltpu.VMEM((DSLOT * CH,), jnp.int32),
            pltpu.VMEM((max(rem, 8),), jnp.int32),
            [pltpu.SemaphoreType.DMA] * DSLOT,
            [pltpu.SemaphoreType.DMA] * DSLOT,
        ],
    )
    def sc_deg(dst_hbm, zeros_hbm, ones_hbm, deg_hbm, dacc, ones_v, idxd,
               rdidx, isem, ssem):
        c = lax.axis_index("c")
        s = lax.axis_index("s")
        base = (c * NS + s) * ept

        def islice(slot):
            return idxd.at[pl.ds(slot * CH, CH)]

        def prefetch(j, slot):
            off = base + jnp.minimum(j, n_full - 1) * CH
            pltpu.async_copy(dst_hbm.at[pl.ds(off, CH)], islice(slot),
                             isem[slot])

        def wait_idx(slot):
            pltpu.make_async_copy(dst_hbm.at[pl.ds(base, CH)], islice(slot),
                                  isem[slot]).wait()

        def scatter(slot):
            pltpu.async_copy(ones_v, dacc.at[islice(slot)], ssem[slot],
                             add=True)

        def drain_scatter(slot):
            pltpu.make_async_copy(ones_hbm, ones_v, ssem[slot]).wait()

        for slot in range(DSLOT):
            prefetch(jnp.int32(slot), slot)
        pltpu.sync_copy(ones_hbm, ones_v)
        _tile_row_copy(s, n, lambda r0, sz: pltpu.sync_copy(
            zeros_hbm.at[pl.ds(0, sz)], dacc.at[pl.ds(r0, sz)]))
        plsc.subcore_barrier()

        half = DSLOT // 2
        for t in range(half):
            wait_idx(t)
            scatter(t)
        for t in range(half, DSLOT):
            drain_scatter((t + half) % DSLOT)
            prefetch(jnp.int32(t + half), (t + half) % DSLOT)
            wait_idx(t)
            scatter(t)

        def body(m, carry):
            jb = DSLOT + m * DSLOT
            for t in range(DSLOT):
                drain_scatter((t + half) % DSLOT)
                prefetch(jb + t + half, (t + half) % DSLOT)
                wait_idx(t)
                scatter(t)
            return carry

        lax.fori_loop(0, (n_full - DSLOT) // DSLOT, body, 0)

        for t in range(half):
            wait_idx(t)
        for t in range(half, DSLOT):
            drain_scatter(t)
        if rem:
            off = base + n_full * CH
            pltpu.sync_copy(dst_hbm.at[pl.ds(off, rem)], rdidx.at[pl.ds(0, rem)])
            pltpu.sync_copy(ones_v.at[pl.ds(0, rem)],
                            dacc.at[rdidx.at[pl.ds(0, rem)]], add=True)
        plsc.subcore_barrier()
        _tile_row_copy(s, n, lambda r0, sz: pltpu.sync_copy(
            dacc.at[pl.ds(r0, sz)], deg_hbm.at[c, pl.ds(r0, sz)]))

    return sc_deg


NB = 3
NSLOT = 2 * NB


def _edge_chunk_counts(e):
    total = e // CH
    lo, extra = divmod(total, NS)
    return lo, extra


@functools.lru_cache(maxsize=None)
def _make_sc_scatter(n, e):
    lo, extra = _edge_chunk_counts(e)
    kpt = lo + (1 if extra else 0)
    main = lo
    assert main % NSLOT == 0
    n_bodies = (main - NSLOT) // NSLOT
    mesh = plsc.VectorSubcoreMesh(core_axis_name="c", subcore_axis_name="s")

    @functools.partial(
        pl.kernel,
        out_type=jax.ShapeDtypeStruct((NC, n, F), jnp.float32),
        mesh=mesh,
        scratch_types=[
            pltpu.VMEM_SHARED((n, F), jnp.float32),
            pltpu.VMEM((NB, CH, F), jnp.float32),
            pltpu.VMEM((2 * NSLOT * CH,), jnp.int32),
            [pltpu.SemaphoreType.DMA] * NB,
            [pltpu.SemaphoreType.DMA] * NB,
            [pltpu.SemaphoreType.DMA] * NSLOT,
        ],
    )
    def sc_scatter(p_hbm, ei_hbm, q_hbm, acc, rows, idxb, gsem, ssem, isem):
        c = lax.axis_index("c")
        s = lax.axis_index("s")

        def islice(slot):
            return idxb.at[pl.ds(2 * slot * CH, 2 * CH)]

        def prefetch(j, slot):
            jj = jnp.minimum(j, kpt - 1)
            pltpu.async_copy(ei_hbm.at[s, jj], islice(slot), isem[slot])

        def wait_idx(slot):
            pltpu.make_async_copy(ei_hbm.at[s, 0], islice(slot),
                                  isem[slot]).wait()

        def gather(slot, b):
            return pltpu.async_copy(
                p_hbm.at[c].at[idxb.at[pl.ds(2 * slot * CH, CH)]],
                rows.at[b], gsem[b])

        def scatter(slot, b):
            pltpu.async_copy(rows.at[b],
                             acc.at[idxb.at[pl.ds((2 * slot + 1) * CH, CH)]],
                             ssem[b], add=True)

        def drain_scatter(b):
            pltpu.make_async_copy(p_hbm.at[c, pl.ds(0, CH)], rows.at[b],
                                  ssem[b]).wait()

        for slot in range(NSLOT):
            prefetch(jnp.int32(slot), slot)
        _tile_row_copy(s, n, lambda r0, sz: pltpu.sync_copy(
            p_hbm.at[c, pl.ds(r0, sz)], acc.at[pl.ds(r0, sz)]))
        plsc.subcore_barrier()

        ds_ = []
        for i in range(NB):
            wait_idx(i)
            ds_.append(gather(i, i))
        for i in range(NB):
            ds_[i].wait()
            scatter(i, i)
        ds_ = []
        for i in range(NB):
            drain_scatter(i)
            prefetch(jnp.int32(NSLOT + i), i)
            wait_idx(NB + i)
            ds_.append(gather(NB + i, i))
        for i in range(NB):
            ds_[i].wait()
            scatter(NB + i, i)

        def body(m, carry):
            jb = NSLOT + m * NSLOT
            for g in range(2):
                ds_ = []
                for i in range(NB):
                    t = g * NB + i
                    drain_scatter(i)
                    prefetch(jb + t + NB, (t + NB) % NSLOT)
                    wait_idx(t)
                    ds_.append(gather(t, i))
                for i in range(NB):
                    ds_[i].wait()
                    scatter(g * NB + i, i)
            return carry

        lax.fori_loop(0, n_bodies, body, 0)

        for i in range(NB):
            drain_scatter(i)
        wait_idx(0)
        if extra:
            @pl.when(s < extra)
            def _():
                pltpu.sync_copy(p_hbm.at[c].at[idxb.at[pl.ds(0, CH)]],
                                rows.at[0])
                pltpu.sync_copy(rows.at[0], acc.at[idxb.at[pl.ds(CH, CH)]],
                                add=True)
        for i in range(1, NB):
            wait_idx(i)

        plsc.subcore_barrier()
        _tile_row_copy(s, n, lambda r0, sz: pltpu.sync_copy(
            acc.at[pl.ds(r0, sz)], q_hbm.at[c, pl.ds(r0, sz)]))

    return sc_scatter


BN = 2000


def _tc_first_body(degp_ref, x_ref, w_ref, dis_ref, p_ref):
    deg = degp_ref[0, :, :1] + degp_ref[1, :, :1] + 1.0
    dis = lax.rsqrt(deg)
    p = jnp.dot(x_ref[...], w_ref[...], preferred_element_type=jnp.float32)
    p = p * dis
    dis_ref[...] = dis
    p_ref[0] = p[:, :F]
    p_ref[1] = p[:, F:]


def _tc_mid_body(q_ref, dis_ref, w_ref, b_ref, p_ref):
    dis = dis_ref[...]
    b = b_ref[...]
    h0 = jnp.tanh(q_ref[0] * dis + b[:, :F])
    h1 = jnp.tanh(q_ref[1] * dis + b[:, F:])
    p = (jnp.dot(h0, w_ref[0], preferred_element_type=jnp.float32)
         + jnp.dot(h1, w_ref[1], preferred_element_type=jnp.float32))
    p = p * dis
    p_ref[0] = p[:, :F]
    p_ref[1] = p[:, F:]


def _tc_last_body(q_ref, dis_ref, w_ref, b_ref, bout_ref, o_ref):
    dis = dis_ref[...]
    b = b_ref[...]
    h0 = jnp.tanh(q_ref[0] * dis + b[:, :F])
    h1 = jnp.tanh(q_ref[1] * dis + b[:, F:])
    o_ref[...] = (jnp.dot(h0, w_ref[0], preferred_element_type=jnp.float32)
                  + jnp.dot(h1, w_ref[1], preferred_element_type=jnp.float32)
                  + bout_ref[...])


def _tc_first(degp, x, w0):
    n, d_in = x.shape
    d_h = w0.shape[1]
    grid = n // BN
    return pl.pallas_call(
        _tc_first_body,
        grid=(grid,),
        in_specs=[
            pl.BlockSpec((NC, BN, F), lambda i: (0, i, 0)),
            pl.BlockSpec((BN, d_in), lambda i: (i, 0)),
            pl.BlockSpec((d_in, d_h), lambda i: (0, 0)),
        ],
        out_specs=[
            pl.BlockSpec((BN, 1), lambda i: (i, 0)),
            pl.BlockSpec((NC, BN, F), lambda i: (0, i, 0)),
        ],
        out_shape=[
            jax.ShapeDtypeStruct((n, 1), jnp.float32),
            jax.ShapeDtypeStruct((NC, n, F), jnp.float32),
        ],
    )(degp, x, w0)


def _tc_mid(q, dis, w, b):
    n = dis.shape[0]
    d_h = w.shape[2]
    grid = n // BN
    return pl.pallas_call(
        _tc_mid_body,
        grid=(grid,),
        in_specs=[
            pl.BlockSpec((NC, BN, F), lambda i: (0, i, 0)),
            pl.BlockSpec((BN, 1), lambda i: (i, 0)),
            pl.BlockSpec((NC, F, d_h), lambda i: (0, 0, 0)),
            pl.BlockSpec((1, 2 * F), lambda i: (0, 0)),
        ],
        out_specs=pl.BlockSpec((NC, BN, F), lambda i: (0, i, 0)),
        out_shape=jax.ShapeDtypeStruct((NC, n, F), jnp.float32),
    )(q, dis, w, b)


def _tc_last(q, dis, w, b, bout):
    n = dis.shape[0]
    d_out = w.shape[2]
    grid = n // BN
    return pl.pallas_call(
        _tc_last_body,
        grid=(grid,),
        in_specs=[
            pl.BlockSpec((NC, BN, F), lambda i: (0, i, 0)),
            pl.BlockSpec((BN, 1), lambda i: (i, 0)),
            pl.BlockSpec((NC, F, d_out), lambda i: (0, 0, 0)),
            pl.BlockSpec((1, 2 * F), lambda i: (0, 0)),
            pl.BlockSpec((1, d_out), lambda i: (0, 0)),
        ],
        out_specs=pl.BlockSpec((BN, d_out), lambda i: (i, 0)),
        out_shape=jax.ShapeDtypeStruct((n, d_out), jnp.float32),
    )(q, dis, w, b, bout)


def kernel(x, edge_index, W0, b0, W1, b1, W2, b2, W3, b3, Wout, bout):
    n = x.shape[0]
    e = edge_index.shape[1]

    sc_deg = _make_sc_deg(n, e)
    sc_scatter = _make_sc_scatter(n, e)

    zeros = jnp.zeros((-(-(n // NS) // 8) * 8, F), jnp.float32)
    ones = jnp.ones((CH, F), jnp.float32)
    src = edge_index[0]
    dst = edge_index[1]

    lo, extra = _edge_chunk_counts(e)

    def chunked(a):
        if not extra:
            return a.reshape(NS, lo, CH)
        p1 = a[:extra * (lo + 1) * CH].reshape(extra, lo + 1, CH)
        p2 = a[extra * (lo + 1) * CH:].reshape(NS - extra, lo, CH)
        pad = jnp.zeros((NS - extra, 1, CH), jnp.int32)
        return jnp.concatenate([p1, jnp.concatenate([p2, pad], axis=1)], axis=0)

    ei4 = jnp.stack([chunked(src), chunked(dst)], axis=2).reshape(
        NS, -1, 2 * CH)

    degp = sc_deg(dst, zeros, ones)
    dis, p = _tc_first(degp, x, W0)

    q = sc_scatter(p, ei4)
    p = _tc_mid(q, dis, W1.reshape(NC, F, -1), b0.reshape(1, -1))
    q = sc_scatter(p, ei4)
    p = _tc_mid(q, dis, W2.reshape(NC, F, -1), b1.reshape(1, -1))
    q = sc_scatter(p, ei4)
    p = _tc_mid(q, dis, W3.reshape(NC, F, -1), b2.reshape(1, -1))
    q = sc_scatter(p, ei4)
    return _tc_last(q, dis, Wout.reshape(NC, F, -1), b3.reshape(1, -1),
                    bout.reshape(1, -1))


# speedup vs baseline: 18.9384x; 1.0434x over previous
import functools

import jax
import jax.numpy as jnp
from jax import lax
from jax.experimental import pallas as pl
from jax.experimental.pallas import tpu as pltpu
from jax.experimental.pallas import tpu_sc as plsc

NC = 2
NS = 16
CH = 64
F = 128


def _tile_row_copy(s, n, copy_fn):
    rpt = -(-(n // NS) // 8) * 8
    last = n - (NS - 1) * rpt
    assert last > 0 and last % 8 == 0

    @pl.when(s < NS - 1)
    def _():
        copy_fn(pl.multiple_of(s * rpt, 8), rpt)

    @pl.when(s == NS - 1)
    def _():
        copy_fn((NS - 1) * rpt, last)


DSLOT = 6


@functools.lru_cache(maxsize=None)
def _make_sc_deg(n, e):
    ept = e // (NC * NS)
    n_full, rem = divmod(ept, CH)
    assert n_full % DSLOT == 0
    mesh = plsc.VectorSubcoreMesh(core_axis_name="c", subcore_axis_name="s")

    @functools.partial(
        pl.kernel,
        out_type=jax.ShapeDtypeStruct((NC, n, F), jnp.float32),
        mesh=mesh,
        scratch_types=[
            pltpu.VMEM_SHARED((n, F), jnp.float32),
            pltpu.VMEM((CH, F), jnp.float32),
            pltpu.VMEM((DSLOT * CH,), jnp.int32),
            pltpu.VMEM((max(rem, 8),), jnp.int32),
            [pltpu.SemaphoreType.DMA] * DSLOT,
            [pltpu.SemaphoreType.DMA] * DSLOT,
        ],
    )
    def sc_deg(dst_hbm, zeros_hbm, ones_hbm, deg_hbm, dacc, ones_v, idxd,
               rdidx, isem, ssem):
        c = lax.axis_index("c")
        s = lax.axis_index("s")
        base = (c * NS + s) * ept

        def islice(slot):
            return idxd.at[pl.ds(slot * CH, CH)]

        def prefetch(j, slot):
            off = base + jnp.minimum(j, n_full - 1) * CH
            pltpu.async_copy(dst_hbm.at[pl.ds(off, CH)], islice(slot),
                             isem[slot])

        def wait_idx(slot):
            pltpu.make_async_copy(dst_hbm.at[pl.ds(base, CH)], islice(slot),
                                  isem[slot]).wait()

        def scatter(slot):
            pltpu.async_copy(ones_v, dacc.at[islice(slot)], ssem[slot],
                             add=True)

        def drain_scatter(slot):
            pltpu.make_async_copy(ones_hbm, ones_v, ssem[slot]).wait()

        for slot in range(DSLOT):
            prefetch(jnp.int32(slot), slot)
        pltpu.sync_copy(ones_hbm, ones_v)
        _tile_row_copy(s, n, lambda r0, sz: pltpu.sync_copy(
            zeros_hbm.at[pl.ds(0, sz)], dacc.at[pl.ds(r0, sz)]))
        plsc.subcore_barrier()

        half = DSLOT // 2
        for t in range(half):
            wait_idx(t)
            scatter(t)
        for t in range(half, DSLOT):
            drain_scatter((t + half) % DSLOT)
            prefetch(jnp.int32(t + half), (t + half) % DSLOT)
            wait_idx(t)
            scatter(t)

        def body(m, carry):
            jb = DSLOT + m * DSLOT
            for t in range(DSLOT):
                drain_scatter((t + half) % DSLOT)
                prefetch(jb + t + half, (t + half) % DSLOT)
                wait_idx(t)
                scatter(t)
            return carry

        lax.fori_loop(0, (n_full - DSLOT) // DSLOT, body, 0)

        for t in range(half):
            wait_idx(t)
        for t in range(half, DSLOT):
            drain_scatter(t)
        if rem:
            off = base + n_full * CH
            pltpu.sync_copy(dst_hbm.at[pl.ds(off, rem)], rdidx.at[pl.ds(0, rem)])
            pltpu.sync_copy(ones_v.at[pl.ds(0, rem)],
                            dacc.at[rdidx.at[pl.ds(0, rem)]], add=True)
        plsc.subcore_barrier()
        _tile_row_copy(s, n, lambda r0, sz: pltpu.sync_copy(
            dacc.at[pl.ds(r0, sz)], deg_hbm.at[c, pl.ds(r0, sz)]))

    return sc_deg


NB = 4
NSLOT = 2 * NB


def _edge_chunk_counts(e):
    total = e // CH
    lo, extra = divmod(total, NS)
    return lo, extra


@functools.lru_cache(maxsize=None)
def _make_sc_scatter(n, e):
    lo, extra = _edge_chunk_counts(e)
    kpt = lo + (1 if extra else 0)
    main = lo
    assert main % NSLOT == 0
    n_bodies = (main - NSLOT) // NSLOT
    mesh = plsc.VectorSubcoreMesh(core_axis_name="c", subcore_axis_name="s")

    @functools.partial(
        pl.kernel,
        out_type=jax.ShapeDtypeStruct((NC, n, F), jnp.float32),
        mesh=mesh,
        scratch_types=[
            pltpu.VMEM_SHARED((n, F), jnp.float32),
            pltpu.VMEM((NB, CH, F), jnp.float32),
            pltpu.VMEM((2 * NSLOT * CH,), jnp.int32),
            [pltpu.SemaphoreType.DMA] * NB,
            [pltpu.SemaphoreType.DMA] * NB,
            [pltpu.SemaphoreType.DMA] * NSLOT,
        ],
    )
    def sc_scatter(p_hbm, ei_hbm, q_hbm, acc, rows, idxb, gsem, ssem, isem):
        c = lax.axis_index("c")
        s = lax.axis_index("s")

        def islice(slot):
            return idxb.at[pl.ds(2 * slot * CH, 2 * CH)]

        def prefetch(j, slot):
            jj = jnp.minimum(j, kpt - 1)
            pltpu.async_copy(ei_hbm.at[s, jj], islice(slot), isem[slot])

        def wait_idx(slot):
            pltpu.make_async_copy(ei_hbm.at[s, 0], islice(slot),
                                  isem[slot]).wait()

        def gather(slot, b):
            return pltpu.async_copy(
                p_hbm.at[c].at[idxb.at[pl.ds(2 * slot * CH, CH)]],
                rows.at[b], gsem[b])

        def scatter(slot, b):
            pltpu.async_copy(rows.at[b],
                             acc.at[idxb.at[pl.ds((2 * slot + 1) * CH, CH)]],
                             ssem[b], add=True)

        def drain_scatter(b):
            pltpu.make_async_copy(p_hbm.at[c, pl.ds(0, CH)], rows.at[b],
                                  ssem[b]).wait()

        for slot in range(NSLOT):
            prefetch(jnp.int32(slot), slot)
        _tile_row_copy(s, n, lambda r0, sz: pltpu.sync_copy(
            p_hbm.at[c, pl.ds(r0, sz)], acc.at[pl.ds(r0, sz)]))
        plsc.subcore_barrier()

        ds_ = []
        for i in range(NB):
            wait_idx(i)
            ds_.append(gather(i, i))
        for i in range(NB):
            ds_[i].wait()
            scatter(i, i)
        ds_ = []
        for i in range(NB):
            drain_scatter(i)
            prefetch(jnp.int32(NSLOT + i), i)
            wait_idx(NB + i)
            ds_.append(gather(NB + i, i))
        for i in range(NB):
            ds_[i].wait()
            scatter(NB + i, i)

        def body(m, carry):
            jb = NSLOT + m * NSLOT
            for g in range(2):
                ds_ = []
                for i in range(NB):
                    t = g * NB + i
                    drain_scatter(i)
                    prefetch(jb + t + NB, (t + NB) % NSLOT)
                    wait_idx(t)
                    ds_.append(gather(t, i))
                for i in range(NB):
                    ds_[i].wait()
                    scatter(g * NB + i, i)
            return carry

        lax.fori_loop(0, n_bodies, body, 0)

        for i in range(NB):
            drain_scatter(i)
        wait_idx(0)
        if extra:
            @pl.when(s < extra)
            def _():
                pltpu.sync_copy(p_hbm.at[c].at[idxb.at[pl.ds(0, CH)]],
                                rows.at[0])
                pltpu.sync_copy(rows.at[0], acc.at[idxb.at[pl.ds(CH, CH)]],
                                add=True)
        for i in range(1, NB):
            wait_idx(i)

        plsc.subcore_barrier()
        _tile_row_copy(s, n, lambda r0, sz: pltpu.sync_copy(
            acc.at[pl.ds(r0, sz)], q_hbm.at[c, pl.ds(r0, sz)]))

    return sc_scatter


BN = 2000


def _tc_first_body(degp_ref, x_ref, w_ref, dis_ref, p_ref):
    deg = degp_ref[0, :, :1] + degp_ref[1, :, :1] + 1.0
    dis = lax.rsqrt(deg)
    p = jnp.dot(x_ref[...], w_ref[...], preferred_element_type=jnp.float32)
    p = p * dis
    dis_ref[...] = dis
    p_ref[0] = p[:, :F]
    p_ref[1] = p[:, F:]


def _tc_mid_body(q_ref, dis_ref, w_ref, b_ref, p_ref):
    dis = dis_ref[...]
    b = b_ref[...]
    h0 = jnp.tanh(q_ref[0] * dis + b[:, :F])
    h1 = jnp.tanh(q_ref[1] * dis + b[:, F:])
    p = (jnp.dot(h0, w_ref[0], preferred_element_type=jnp.float32)
         + jnp.dot(h1, w_ref[1], preferred_element_type=jnp.float32))
    p = p * dis
    p_ref[0] = p[:, :F]
    p_ref[1] = p[:, F:]


def _tc_last_body(q_ref, dis_ref, w_ref, b_ref, bout_ref, o_ref):
    dis = dis_ref[...]
    b = b_ref[...]
    h0 = jnp.tanh(q_ref[0] * dis + b[:, :F])
    h1 = jnp.tanh(q_ref[1] * dis + b[:, F:])
    o_ref[...] = (jnp.dot(h0, w_ref[0], preferred_element_type=jnp.float32)
                  + jnp.dot(h1, w_ref[1], preferred_element_type=jnp.float32)
                  + bout_ref[...])


def _tc_first(degp, x, w0):
    n, d_in = x.shape
    d_h = w0.shape[1]
    grid = n // BN
    return pl.pallas_call(
        _tc_first_body,
        grid=(grid,),
        in_specs=[
            pl.BlockSpec((NC, BN, F), lambda i: (0, i, 0)),
            pl.BlockSpec((BN, d_in), lambda i: (i, 0)),
            pl.BlockSpec((d_in, d_h), lambda i: (0, 0)),
        ],
        out_specs=[
            pl.BlockSpec((BN, 1), lambda i: (i, 0)),
            pl.BlockSpec((NC, BN, F), lambda i: (0, i, 0)),
        ],
        out_shape=[
            jax.ShapeDtypeStruct((n, 1), jnp.float32),
            jax.ShapeDtypeStruct((NC, n, F), jnp.float32),
        ],
    )(degp, x, w0)


def _tc_mid(q, dis, w, b):
    n = dis.shape[0]
    d_h = w.shape[2]
    grid = n // BN
    return pl.pallas_call(
        _tc_mid_body,
        grid=(grid,),
        in_specs=[
            pl.BlockSpec((NC, BN, F), lambda i: (0, i, 0)),
            pl.BlockSpec((BN, 1), lambda i: (i, 0)),
            pl.BlockSpec((NC, F, d_h), lambda i: (0, 0, 0)),
            pl.BlockSpec((1, 2 * F), lambda i: (0, 0)),
        ],
        out_specs=pl.BlockSpec((NC, BN, F), lambda i: (0, i, 0)),
        out_shape=jax.ShapeDtypeStruct((NC, n, F), jnp.float32),
    )(q, dis, w, b)


def _tc_last(q, dis, w, b, bout):
    n = dis.shape[0]
    d_out = w.shape[2]
    grid = n // BN
    return pl.pallas_call(
        _tc_last_body,
        grid=(grid,),
        in_specs=[
            pl.BlockSpec((NC, BN, F), lambda i: (0, i, 0)),
            pl.BlockSpec((BN, 1), lambda i: (i, 0)),
            pl.BlockSpec((NC, F, d_out), lambda i: (0, 0, 0)),
            pl.BlockSpec((1, 2 * F), lambda i: (0, 0)),
            pl.BlockSpec((1, d_out), lambda i: (0, 0)),
        ],
        out_specs=pl.BlockSpec((BN, d_out), lambda i: (i, 0)),
        out_shape=jax.ShapeDtypeStruct((n, d_out), jnp.float32),
    )(q, dis, w, b, bout)


def kernel(x, edge_index, W0, b0, W1, b1, W2, b2, W3, b3, Wout, bout):
    n = x.shape[0]
    e = edge_index.shape[1]

    sc_deg = _make_sc_deg(n, e)
    sc_scatter = _make_sc_scatter(n, e)

    zeros = jnp.zeros((-(-(n // NS) // 8) * 8, F), jnp.float32)
    ones = jnp.ones((CH, F), jnp.float32)
    src = edge_index[0]
    dst = edge_index[1]

    lo, extra = _edge_chunk_counts(e)

    def chunked(a):
        if not extra:
            return a.reshape(NS, lo, CH)
        p1 = a[:extra * (lo + 1) * CH].reshape(extra, lo + 1, CH)
        p2 = a[extra * (lo + 1) * CH:].reshape(NS - extra, lo, CH)
        pad = jnp.zeros((NS - extra, 1, CH), jnp.int32)
        return jnp.concatenate([p1, jnp.concatenate([p2, pad], axis=1)], axis=0)

    ei4 = jnp.stack([chunked(src), chunked(dst)], axis=2).reshape(
        NS, -1, 2 * CH)

    degp = sc_deg(dst, zeros, ones)
    dis, p = _tc_first(degp, x, W0)

    q = sc_scatter(p, ei4)
    p = _tc_mid(q, dis, W1.reshape(NC, F, -1), b0.reshape(1, -1))
    q = sc_scatter(p, ei4)
    p = _tc_mid(q, dis, W2.reshape(NC, F, -1), b1.reshape(1, -1))
    q = sc_scatter(p, ei4)
    p = _tc_mid(q, dis, W3.reshape(NC, F, -1), b2.reshape(1, -1))
    q = sc_scatter(p, ei4)
    return _tc_last(q, dis, Wout.reshape(NC, F, -1), b3.reshape(1, -1),
                    bout.reshape(1, -1))


# speedup vs baseline: 19.9665x; 1.0543x over previous
import functools

import jax
import jax.numpy as jnp
from jax import lax
from jax.experimental import pallas as pl
from jax.experimental.pallas import tpu as pltpu
from jax.experimental.pallas import tpu_sc as plsc

NC = 2
NS = 16
CH = 64
F = 128


def _tile_row_copy(s, n, copy_fn):
    rpt = -(-(n // NS) // 8) * 8
    last = n - (NS - 1) * rpt
    assert last > 0 and last % 8 == 0

    @pl.when(s < NS - 1)
    def _():
        copy_fn(pl.multiple_of(s * rpt, 8), rpt)

    @pl.when(s == NS - 1)
    def _():
        copy_fn((NS - 1) * rpt, last)


DSLOT = 6


@functools.lru_cache(maxsize=None)
def _make_sc_deg(n, e):
    ept = e // (NC * NS)
    n_full, rem = divmod(ept, CH)
    assert n_full % DSLOT == 0
    mesh = plsc.VectorSubcoreMesh(core_axis_name="c", subcore_axis_name="s")

    @functools.partial(
        pl.kernel,
        out_type=jax.ShapeDtypeStruct((NC, n, F), jnp.float32),
        mesh=mesh,
        scratch_types=[
            pltpu.VMEM_SHARED((n, F), jnp.float32),
            pltpu.VMEM((CH, F), jnp.float32),
            pltpu.VMEM((DSLOT * CH,), jnp.int32),
            pltpu.VMEM((max(rem, 8),), jnp.int32),
            [pltpu.SemaphoreType.DMA] * DSLOT,
            [pltpu.SemaphoreType.DMA] * DSLOT,
        ],
    )
    def sc_deg(dst_hbm, zeros_hbm, ones_hbm, deg_hbm, dacc, ones_v, idxd,
               rdidx, isem, ssem):
        c = lax.axis_index("c")
        s = lax.axis_index("s")
        base = (c * NS + s) * ept

        def islice(slot):
            return idxd.at[pl.ds(slot * CH, CH)]

        def prefetch(j, slot):
            off = base + jnp.minimum(j, n_full - 1) * CH
            pltpu.async_copy(dst_hbm.at[pl.ds(off, CH)], islice(slot),
                             isem[slot])

        def wait_idx(slot):
            pltpu.make_async_copy(dst_hbm.at[pl.ds(base, CH)], islice(slot),
                                  isem[slot]).wait()

        def scatter(slot):
            pltpu.async_copy(ones_v, dacc.at[islice(slot)], ssem[slot],
                             add=True)

        def drain_scatter(slot):
            pltpu.make_async_copy(ones_hbm, ones_v, ssem[slot]).wait()

        for slot in range(DSLOT):
            prefetch(jnp.int32(slot), slot)
        pltpu.sync_copy(ones_hbm, ones_v)
        _tile_row_copy(s, n, lambda r0, sz: pltpu.sync_copy(
            zeros_hbm.at[pl.ds(0, sz)], dacc.at[pl.ds(r0, sz)]))
        plsc.subcore_barrier()

        half = DSLOT // 2
        for t in range(half):
            wait_idx(t)
            scatter(t)
        for t in range(half, DSLOT):
            drain_scatter((t + half) % DSLOT)
            prefetch(jnp.int32(t + half), (t + half) % DSLOT)
            wait_idx(t)
            scatter(t)

        def body(m, carry):
            jb = DSLOT + m * DSLOT
            for t in range(DSLOT):
                drain_scatter((t + half) % DSLOT)
                prefetch(jb + t + half, (t + half) % DSLOT)
                wait_idx(t)
                scatter(t)
            return carry

        lax.fori_loop(0, (n_full - DSLOT) // DSLOT, body, 0)

        for t in range(half):
            wait_idx(t)
        for t in range(half, DSLOT):
            drain_scatter(t)
        if rem:
            off = base + n_full * CH
            pltpu.sync_copy(dst_hbm.at[pl.ds(off, rem)], rdidx.at[pl.ds(0, rem)])
            pltpu.sync_copy(ones_v.at[pl.ds(0, rem)],
                            dacc.at[rdidx.at[pl.ds(0, rem)]], add=True)
        plsc.subcore_barrier()
        _tile_row_copy(s, n, lambda r0, sz: pltpu.sync_copy(
            dacc.at[pl.ds(r0, sz)], deg_hbm.at[c, pl.ds(r0, sz)]))

    return sc_deg


NB = 6
NSLOT = 2 * NB


def _edge_chunk_counts(e):
    total = e // CH
    lo, extra = divmod(total, NS)
    return lo, extra


@functools.lru_cache(maxsize=None)
def _make_sc_scatter(n, e):
    lo, extra = _edge_chunk_counts(e)
    kpt = lo + (1 if extra else 0)
    main = lo
    assert main % NSLOT == 0
    n_bodies = (main - NSLOT) // NSLOT
    mesh = plsc.VectorSubcoreMesh(core_axis_name="c", subcore_axis_name="s")

    @functools.partial(
        pl.kernel,
        out_type=jax.ShapeDtypeStruct((NC, n, F), jnp.float32),
        mesh=mesh,
        scratch_types=[
            pltpu.VMEM_SHARED((n, F), jnp.float32),
            pltpu.VMEM((NB, CH, F), jnp.float32),
            pltpu.VMEM((2 * NSLOT * CH,), jnp.int32),
            [pltpu.SemaphoreType.DMA] * NB,
            [pltpu.SemaphoreType.DMA] * NB,
            [pltpu.SemaphoreType.DMA] * NSLOT,
        ],
    )
    def sc_scatter(p_hbm, ei_hbm, q_hbm, acc, rows, idxb, gsem, ssem, isem):
        c = lax.axis_index("c")
        s = lax.axis_index("s")

        def islice(slot):
            return idxb.at[pl.ds(2 * slot * CH, 2 * CH)]

        def prefetch(j, slot):
            jj = jnp.minimum(j, kpt - 1)
            pltpu.async_copy(ei_hbm.at[s, jj], islice(slot), isem[slot])

        def wait_idx(slot):
            pltpu.make_async_copy(ei_hbm.at[s, 0], islice(slot),
                                  isem[slot]).wait()

        def gather(slot, b):
            return pltpu.async_copy(
                p_hbm.at[c].at[idxb.at[pl.ds(2 * slot * CH, CH)]],
                rows.at[b], gsem[b])

        def scatter(slot, b):
            pltpu.async_copy(rows.at[b],
                             acc.at[idxb.at[pl.ds((2 * slot + 1) * CH, CH)]],
                             ssem[b], add=True)

        def drain_scatter(b):
            pltpu.make_async_copy(p_hbm.at[c, pl.ds(0, CH)], rows.at[b],
                                  ssem[b]).wait()

        for slot in range(NSLOT):
            prefetch(jnp.int32(slot), slot)
        _tile_row_copy(s, n, lambda r0, sz: pltpu.sync_copy(
            p_hbm.at[c, pl.ds(r0, sz)], acc.at[pl.ds(r0, sz)]))
        plsc.subcore_barrier()

        ds_ = []
        for i in range(NB):
            wait_idx(i)
            ds_.append(gather(i, i))
        for i in range(NB):
            ds_[i].wait()
            scatter(i, i)
        ds_ = []
        for i in range(NB):
            drain_scatter(i)
            prefetch(jnp.int32(NSLOT + i), i)
            wait_idx(NB + i)
            ds_.append(gather(NB + i, i))
        for i in range(NB):
            ds_[i].wait()
            scatter(NB + i, i)

        def body(m, carry):
            jb = NSLOT + m * NSLOT
            for g in range(2):
                ds_ = []
                for i in range(NB):
                    t = g * NB + i
                    drain_scatter(i)
                    prefetch(jb + t + NB, (t + NB) % NSLOT)
                    wait_idx(t)
                    ds_.append(gather(t, i))
                for i in range(NB):
                    ds_[i].wait()
                    scatter(g * NB + i, i)
            return carry

        lax.fori_loop(0, n_bodies, body, 0)

        for i in range(NB):
            drain_scatter(i)
        wait_idx(0)
        if extra:
            @pl.when(s < extra)
            def _():
                pltpu.sync_copy(p_hbm.at[c].at[idxb.at[pl.ds(0, CH)]],
                                rows.at[0])
                pltpu.sync_copy(rows.at[0], acc.at[idxb.at[pl.ds(CH, CH)]],
                                add=True)
        for i in range(1, NB):
            wait_idx(i)

        plsc.subcore_barrier()
        _tile_row_copy(s, n, lambda r0, sz: pltpu.sync_copy(
            acc.at[pl.ds(r0, sz)], q_hbm.at[c, pl.ds(r0, sz)]))

    return sc_scatter


BN = 2000


def _tc_first_body(degp_ref, x_ref, w_ref, dis_ref, p_ref):
    deg = degp_ref[0, :, :1] + degp_ref[1, :, :1] + 1.0
    dis = lax.rsqrt(deg)
    p = jnp.dot(x_ref[...], w_ref[...], preferred_element_type=jnp.float32)
    p = p * dis
    dis_ref[...] = dis
    p_ref[0] = p[:, :F]
    p_ref[1] = p[:, F:]


def _tc_mid_body(q_ref, dis_ref, w_ref, b_ref, p_ref):
    dis = dis_ref[...]
    b = b_ref[...]
    h0 = jnp.tanh(q_ref[0] * dis + b[:, :F])
    h1 = jnp.tanh(q_ref[1] * dis + b[:, F:])
    p = (jnp.dot(h0, w_ref[0], preferred_element_type=jnp.float32)
         + jnp.dot(h1, w_ref[1], preferred_element_type=jnp.float32))
    p = p * dis
    p_ref[0] = p[:, :F]
    p_ref[1] = p[:, F:]


def _tc_last_body(q_ref, dis_ref, w_ref, b_ref, bout_ref, o_ref):
    dis = dis_ref[...]
    b = b_ref[...]
    h0 = jnp.tanh(q_ref[0] * dis + b[:, :F])
    h1 = jnp.tanh(q_ref[1] * dis + b[:, F:])
    o_ref[...] = (jnp.dot(h0, w_ref[0], preferred_element_type=jnp.float32)
                  + jnp.dot(h1, w_ref[1], preferred_element_type=jnp.float32)
                  + bout_ref[...])


def _tc_first(degp, x, w0):
    n, d_in = x.shape
    d_h = w0.shape[1]
    grid = n // BN
    return pl.pallas_call(
        _tc_first_body,
        grid=(grid,),
        in_specs=[
            pl.BlockSpec((NC, BN, F), lambda i: (0, i, 0)),
            pl.BlockSpec((BN, d_in), lambda i: (i, 0)),
            pl.BlockSpec((d_in, d_h), lambda i: (0, 0)),
        ],
        out_specs=[
            pl.BlockSpec((BN, 1), lambda i: (i, 0)),
            pl.BlockSpec((NC, BN, F), lambda i: (0, i, 0)),
        ],
        out_shape=[
            jax.ShapeDtypeStruct((n, 1), jnp.float32),
            jax.ShapeDtypeStruct((NC, n, F), jnp.float32),
        ],
    )(degp, x, w0)


def _tc_mid(q, dis, w, b):
    n = dis.shape[0]
    d_h = w.shape[2]
    grid = n // BN
    return pl.pallas_call(
        _tc_mid_body,
        grid=(grid,),
        in_specs=[
            pl.BlockSpec((NC, BN, F), lambda i: (0, i, 0)),
            pl.BlockSpec((BN, 1), lambda i: (i, 0)),
            pl.BlockSpec((NC, F, d_h), lambda i: (0, 0, 0)),
            pl.BlockSpec((1, 2 * F), lambda i: (0, 0)),
        ],
        out_specs=pl.BlockSpec((NC, BN, F), lambda i: (0, i, 0)),
        out_shape=jax.ShapeDtypeStruct((NC, n, F), jnp.float32),
    )(q, dis, w, b)


def _tc_last(q, dis, w, b, bout):
    n = dis.shape[0]
    d_out = w.shape[2]
    grid = n // BN
    return pl.pallas_call(
        _tc_last_body,
        grid=(grid,),
        in_specs=[
            pl.BlockSpec((NC, BN, F), lambda i: (0, i, 0)),
            pl.BlockSpec((BN, 1), lambda i: (i, 0)),
            pl.BlockSpec((NC, F, d_out), lambda i: (0, 0, 0)),
            pl.BlockSpec((1, 2 * F), lambda i: (0, 0)),
            pl.BlockSpec((1, d_out), lambda i: (0, 0)),
        ],
        out_specs=pl.BlockSpec((BN, d_out), lambda i: (i, 0)),
        out_shape=jax.ShapeDtypeStruct((n, d_out), jnp.float32),
    )(q, dis, w, b, bout)


def kernel(x, edge_index, W0, b0, W1, b1, W2, b2, W3, b3, Wout, bout):
    n = x.shape[0]
    e = edge_index.shape[1]

    sc_deg = _make_sc_deg(n, e)
    sc_scatter = _make_sc_scatter(n, e)

    zeros = jnp.zeros((-(-(n // NS) // 8) * 8, F), jnp.float32)
    ones = jnp.ones((CH, F), jnp.float32)
    src = edge_index[0]
    dst = edge_index[1]

    lo, extra = _edge_chunk_counts(e)

    def chunked(a):
        if not extra:
            return a.reshape(NS, lo, CH)
        p1 = a[:extra * (lo + 1) * CH].reshape(extra, lo + 1, CH)
        p2 = a[extra * (lo + 1) * CH:].reshape(NS - extra, lo, CH)
        pad = jnp.zeros((NS - extra, 1, CH), jnp.int32)
        return jnp.concatenate([p1, jnp.concatenate([p2, pad], axis=1)], axis=0)

    ei4 = jnp.stack([chunked(src), chunked(dst)], axis=2).reshape(
        NS, -1, 2 * CH)

    degp = sc_deg(dst, zeros, ones)
    dis, p = _tc_first(degp, x, W0)

    q = sc_scatter(p, ei4)
    p = _tc_mid(q, dis, W1.reshape(NC, F, -1), b0.reshape(1, -1))
    q = sc_scatter(p, ei4)
    p = _tc_mid(q, dis, W2.reshape(NC, F, -1), b1.reshape(1, -1))
    q = sc_scatter(p, ei4)
    p = _tc_mid(q, dis, W3.reshape(NC, F, -1), b2.reshape(1, -1))
    q = sc_scatter(p, ei4)
    return _tc_last(q, dis, Wout.reshape(NC, F, -1), b3.reshape(1, -1),
                    bout.reshape(1, -1))


# speedup vs baseline: 19.9850x; 1.0009x over previous
import functools

import jax
import jax.numpy as jnp
from jax import lax
from jax.experimental import pallas as pl
from jax.experimental.pallas import tpu as pltpu
from jax.experimental.pallas import tpu_sc as plsc

NC = 2
NS = 16
CH = 64
F = 128


def _tile_row_copy(s, n, copy_fn):
    rpt = -(-(n // NS) // 8) * 8
    last = n - (NS - 1) * rpt
    assert last > 0 and last % 8 == 0

    @pl.when(s < NS - 1)
    def _():
        copy_fn(pl.multiple_of(s * rpt, 8), rpt)

    @pl.when(s == NS - 1)
    def _():
        copy_fn((NS - 1) * rpt, last)


DSLOT = 12


@functools.lru_cache(maxsize=None)
def _make_sc_deg(n, e):
    ept = e // (NC * NS)
    n_full, rem = divmod(ept, CH)
    assert n_full % DSLOT == 0
    mesh = plsc.VectorSubcoreMesh(core_axis_name="c", subcore_axis_name="s")

    @functools.partial(
        pl.kernel,
        out_type=jax.ShapeDtypeStruct((NC, n, F), jnp.float32),
        mesh=mesh,
        scratch_types=[
            pltpu.VMEM_SHARED((n, F), jnp.float32),
            pltpu.VMEM((CH, F), jnp.float32),
            pltpu.VMEM((DSLOT * CH,), jnp.int32),
            pltpu.VMEM((max(rem, 8),), jnp.int32),
            [pltpu.SemaphoreType.DMA] * DSLOT,
            [pltpu.SemaphoreType.DMA] * DSLOT,
        ],
    )
    def sc_deg(dst_hbm, zeros_hbm, ones_hbm, deg_hbm, dacc, ones_v, idxd,
               rdidx, isem, ssem):
        c = lax.axis_index("c")
        s = lax.axis_index("s")
        base = (c * NS + s) * ept

        def islice(slot):
            return idxd.at[pl.ds(slot * CH, CH)]

        def prefetch(j, slot):
            off = base + jnp.minimum(j, n_full - 1) * CH
            pltpu.async_copy(dst_hbm.at[pl.ds(off, CH)], islice(slot),
                             isem[slot])

        def wait_idx(slot):
            pltpu.make_async_copy(dst_hbm.at[pl.ds(base, CH)], islice(slot),
                                  isem[slot]).wait()

        def scatter(slot):
            pltpu.async_copy(ones_v, dacc.at[islice(slot)], ssem[slot],
                             add=True)

        def drain_scatter(slot):
            pltpu.make_async_copy(ones_hbm, ones_v, ssem[slot]).wait()

        for slot in range(DSLOT):
            prefetch(jnp.int32(slot), slot)
        pltpu.sync_copy(ones_hbm, ones_v)
        _tile_row_copy(s, n, lambda r0, sz: pltpu.sync_copy(
            zeros_hbm.at[pl.ds(0, sz)], dacc.at[pl.ds(r0, sz)]))
        plsc.subcore_barrier()

        half = DSLOT // 2
        for t in range(half):
            wait_idx(t)
            scatter(t)
        for t in range(half, DSLOT):
            drain_scatter((t + half) % DSLOT)
            prefetch(jnp.int32(t + half), (t + half) % DSLOT)
            wait_idx(t)
            scatter(t)

        def body(m, carry):
            jb = DSLOT + m * DSLOT
            for t in range(DSLOT):
                drain_scatter((t + half) % DSLOT)
                prefetch(jb + t + half, (t + half) % DSLOT)
                wait_idx(t)
                scatter(t)
            return carry

        lax.fori_loop(0, (n_full - DSLOT) // DSLOT, body, 0)

        for t in range(half):
            wait_idx(t)
        for t in range(half, DSLOT):
            drain_scatter(t)
        if rem:
            off = base + n_full * CH
            pltpu.sync_copy(dst_hbm.at[pl.ds(off, rem)], rdidx.at[pl.ds(0, rem)])
            pltpu.sync_copy(ones_v.at[pl.ds(0, rem)],
                            dacc.at[rdidx.at[pl.ds(0, rem)]], add=True)
        plsc.subcore_barrier()
        _tile_row_copy(s, n, lambda r0, sz: pltpu.sync_copy(
            dacc.at[pl.ds(r0, sz)], deg_hbm.at[c, pl.ds(r0, sz)]))

    return sc_deg


NB = 6
NSLOT = 2 * NB


def _edge_chunk_counts(e):
    total = e // CH
    lo, extra = divmod(total, NS)
    return lo, extra


@functools.lru_cache(maxsize=None)
def _make_sc_scatter(n, e):
    lo, extra = _edge_chunk_counts(e)
    kpt = lo + (1 if extra else 0)
    main = lo
    assert main % NSLOT == 0
    n_bodies = (main - NSLOT) // NSLOT
    mesh = plsc.VectorSubcoreMesh(core_axis_name="c", subcore_axis_name="s")

    @functools.partial(
        pl.kernel,
        out_type=jax.ShapeDtypeStruct((NC, n, F), jnp.float32),
        mesh=mesh,
        scratch_types=[
            pltpu.VMEM_SHARED((n, F), jnp.float32),
            pltpu.VMEM((NB, CH, F), jnp.float32),
            pltpu.VMEM((2 * NSLOT * CH,), jnp.int32),
            [pltpu.SemaphoreType.DMA] * NB,
            [pltpu.SemaphoreType.DMA] * NB,
            [pltpu.SemaphoreType.DMA] * NSLOT,
        ],
    )
    def sc_scatter(p_hbm, ei_hbm, q_hbm, acc, rows, idxb, gsem, ssem, isem):
        c = lax.axis_index("c")
        s = lax.axis_index("s")

        def islice(slot):
            return idxb.at[pl.ds(2 * slot * CH, 2 * CH)]

        def prefetch(j, slot):
            jj = jnp.minimum(j, kpt - 1)
            pltpu.async_copy(ei_hbm.at[s, jj], islice(slot), isem[slot])

        def wait_idx(slot):
            pltpu.make_async_copy(ei_hbm.at[s, 0], islice(slot),
                                  isem[slot]).wait()

        def gather(slot, b):
            return pltpu.async_copy(
                p_hbm.at[c].at[idxb.at[pl.ds(2 * slot * CH, CH)]],
                rows.at[b], gsem[b])

        def scatter(slot, b):
            pltpu.async_copy(rows.at[b],
                             acc.at[idxb.at[pl.ds((2 * slot + 1) * CH, CH)]],
                             ssem[b], add=True)

        def drain_scatter(b):
            pltpu.make_async_copy(p_hbm.at[c, pl.ds(0, CH)], rows.at[b],
                                  ssem[b]).wait()

        for slot in range(NSLOT):
            prefetch(jnp.int32(slot), slot)
        _tile_row_copy(s, n, lambda r0, sz: pltpu.sync_copy(
            p_hbm.at[c, pl.ds(r0, sz)], acc.at[pl.ds(r0, sz)]))
        plsc.subcore_barrier()

        ds_ = []
        for i in range(NB):
            wait_idx(i)
            ds_.append(gather(i, i))
        for i in range(NB):
            ds_[i].wait()
            scatter(i, i)
        ds_ = []
        for i in range(NB):
            drain_scatter(i)
            prefetch(jnp.int32(NSLOT + i), i)
            wait_idx(NB + i)
            ds_.append(gather(NB + i, i))
        for i in range(NB):
            ds_[i].wait()
            scatter(NB + i, i)

        def body(m, carry):
            jb = NSLOT + m * NSLOT
            for g in range(2):
                ds_ = []
                for i in range(NB):
                    t = g * NB + i
                    drain_scatter(i)
                    prefetch(jb + t + NB, (t + NB) % NSLOT)
                    wait_idx(t)
                    ds_.append(gather(t, i))
                for i in range(NB):
                    ds_[i].wait()
                    scatter(g * NB + i, i)
            return carry

        lax.fori_loop(0, n_bodies, body, 0)

        for i in range(NB):
            drain_scatter(i)
        wait_idx(0)
        if extra:
            @pl.when(s < extra)
            def _():
                pltpu.sync_copy(p_hbm.at[c].at[idxb.at[pl.ds(0, CH)]],
                                rows.at[0])
                pltpu.sync_copy(rows.at[0], acc.at[idxb.at[pl.ds(CH, CH)]],
                                add=True)
        for i in range(1, NB):
            wait_idx(i)

        plsc.subcore_barrier()
        _tile_row_copy(s, n, lambda r0, sz: pltpu.sync_copy(
            acc.at[pl.ds(r0, sz)], q_hbm.at[c, pl.ds(r0, sz)]))

    return sc_scatter


BN = 2000


def _tc_first_body(degp_ref, x_ref, w_ref, dis_ref, p_ref):
    deg = degp_ref[0, :, :1] + degp_ref[1, :, :1] + 1.0
    dis = lax.rsqrt(deg)
    p = jnp.dot(x_ref[...], w_ref[...], preferred_element_type=jnp.float32)
    p = p * dis
    dis_ref[...] = dis
    p_ref[0] = p[:, :F]
    p_ref[1] = p[:, F:]


def _tc_mid_body(q_ref, dis_ref, w_ref, b_ref, p_ref):
    dis = dis_ref[...]
    b = b_ref[...]
    h0 = jnp.tanh(q_ref[0] * dis + b[:, :F])
    h1 = jnp.tanh(q_ref[1] * dis + b[:, F:])
    p = (jnp.dot(h0, w_ref[0], preferred_element_type=jnp.float32)
         + jnp.dot(h1, w_ref[1], preferred_element_type=jnp.float32))
    p = p * dis
    p_ref[0] = p[:, :F]
    p_ref[1] = p[:, F:]


def _tc_last_body(q_ref, dis_ref, w_ref, b_ref, bout_ref, o_ref):
    dis = dis_ref[...]
    b = b_ref[...]
    h0 = jnp.tanh(q_ref[0] * dis + b[:, :F])
    h1 = jnp.tanh(q_ref[1] * dis + b[:, F:])
    o_ref[...] = (jnp.dot(h0, w_ref[0], preferred_element_type=jnp.float32)
                  + jnp.dot(h1, w_ref[1], preferred_element_type=jnp.float32)
                  + bout_ref[...])


def _tc_first(degp, x, w0):
    n, d_in = x.shape
    d_h = w0.shape[1]
    grid = n // BN
    return pl.pallas_call(
        _tc_first_body,
        grid=(grid,),
        in_specs=[
            pl.BlockSpec((NC, BN, F), lambda i: (0, i, 0)),
            pl.BlockSpec((BN, d_in), lambda i: (i, 0)),
            pl.BlockSpec((d_in, d_h), lambda i: (0, 0)),
        ],
        out_specs=[
            pl.BlockSpec((BN, 1), lambda i: (i, 0)),
            pl.BlockSpec((NC, BN, F), lambda i: (0, i, 0)),
        ],
        out_shape=[
            jax.ShapeDtypeStruct((n, 1), jnp.float32),
            jax.ShapeDtypeStruct((NC, n, F), jnp.float32),
        ],
    )(degp, x, w0)


def _tc_mid(q, dis, w, b):
    n = dis.shape[0]
    d_h = w.shape[2]
    grid = n // BN
    return pl.pallas_call(
        _tc_mid_body,
        grid=(grid,),
        in_specs=[
            pl.BlockSpec((NC, BN, F), lambda i: (0, i, 0)),
            pl.BlockSpec((BN, 1), lambda i: (i, 0)),
            pl.BlockSpec((NC, F, d_h), lambda i: (0, 0, 0)),
            pl.BlockSpec((1, 2 * F), lambda i: (0, 0)),
        ],
        out_specs=pl.BlockSpec((NC, BN, F), lambda i: (0, i, 0)),
        out_shape=jax.ShapeDtypeStruct((NC, n, F), jnp.float32),
    )(q, dis, w, b)


def _tc_last(q, dis, w, b, bout):
    n = dis.shape[0]
    d_out = w.shape[2]
    grid = n // BN
    return pl.pallas_call(
        _tc_last_body,
        grid=(grid,),
        in_specs=[
            pl.BlockSpec((NC, BN, F), lambda i: (0, i, 0)),
            pl.BlockSpec((BN, 1), lambda i: (i, 0)),
            pl.BlockSpec((NC, F, d_out), lambda i: (0, 0, 0)),
            pl.BlockSpec((1, 2 * F), lambda i: (0, 0)),
            pl.BlockSpec((1, d_out), lambda i: (0, 0)),
        ],
        out_specs=pl.BlockSpec((BN, d_out), lambda i: (i, 0)),
        out_shape=jax.ShapeDtypeStruct((n, d_out), jnp.float32),
    )(q, dis, w, b, bout)


def kernel(x, edge_index, W0, b0, W1, b1, W2, b2, W3, b3, Wout, bout):
    n = x.shape[0]
    e = edge_index.shape[1]

    sc_deg = _make_sc_deg(n, e)
    sc_scatter = _make_sc_scatter(n, e)

    zeros = jnp.zeros((-(-(n // NS) // 8) * 8, F), jnp.float32)
    ones = jnp.ones((CH, F), jnp.float32)
    src = edge_index[0]
    dst = edge_index[1]

    lo, extra = _edge_chunk_counts(e)

    def chunked(a):
        if not extra:
            return a.reshape(NS, lo, CH)
        p1 = a[:extra * (lo + 1) * CH].reshape(extra, lo + 1, CH)
        p2 = a[extra * (lo + 1) * CH:].reshape(NS - extra, lo, CH)
        pad = jnp.zeros((NS - extra, 1, CH), jnp.int32)
        return jnp.concatenate([p1, jnp.concatenate([p2, pad], axis=1)], axis=0)

    ei4 = jnp.stack([chunked(src), chunked(dst)], axis=2).reshape(
        NS, -1, 2 * CH)

    degp = sc_deg(dst, zeros, ones)
    dis, p = _tc_first(degp, x, W0)

    q = sc_scatter(p, ei4)
    p = _tc_mid(q, dis, W1.reshape(NC, F, -1), b0.reshape(1, -1))
    q = sc_scatter(p, ei4)
    p = _tc_mid(q, dis, W2.reshape(NC, F, -1), b1.reshape(1, -1))
    q = sc_scatter(p, ei4)
    p = _tc_mid(q, dis, W3.reshape(NC, F, -1), b2.reshape(1, -1))
    q = sc_scatter(p, ei4)
    return _tc_last(q, dis, Wout.reshape(NC, F, -1), b3.reshape(1, -1),
                    bout.reshape(1, -1))
